# baseline (speedup 1.0000x reference)
import functools

import jax
import jax.numpy as jnp
from jax.experimental import pallas as pl
from jax.experimental.pallas import tpu as pltpu
from jax.experimental.pallas import tpu_sc as plsc

_NE = 8
_DM = 768
_DF = 3072
_SCALE = 0.8
_SPARSITY = 0.1
_T = 4096
_TB = 256

_ABS_MASK = 0x7FFFFFFF
_NW = _DM * _DF
_KW = max(1, int(_NW * _SPARSITY / 2))
_NTILE = 16
_CHUNK = 8192
_PER_TILE = _NW // _NTILE
_NCHUNK = _PER_TILE // _CHUNK
_HSTRIDE = 4096


def _sc_select_body(w1_ref, w2_ref, out_ref,
                    buf, hist, pub, tmp, tmp16, mcs, mraw, small, rr,
                    spub, stot0, stot1, sresp, sresr, sem0, sem1):
    cid = jax.lax.axis_index("c")
    sid = jax.lax.axis_index("s")
    lanes = jax.lax.broadcasted_iota(jnp.int32, (16,), 0)
    ones = jnp.ones((16,), jnp.int32)
    zeros16 = jnp.zeros((16,), jnp.int32)
    sems = (sem0, sem1)

    def scan_ref(w_ref, shift, mask, pshift, pre_b, pre_t, off0, off1, first):
        @plsc.parallel_loop(0, 65536 // 16, 1, unroll=8)
        def _(i):
            hist[pl.ds(i * 16, 16)] = zeros16
        base = sid * _PER_TILE

        def chunk_slice(k):
            return w_ref.at[pl.ds(base + k * _CHUNK, _CHUNK)]

        def process(cur):
            @plsc.parallel_loop(0, _CHUNK // 16, 1, unroll=16)
            def _(g):
                v = buf[cur, pl.ds(g * 16, 16)]
                bits = jax.lax.bitcast_convert_type(v, jnp.int32) & _ABS_MASK
                digit = jax.lax.shift_right_logical(bits, shift) & mask
                if first:
                    plsc.addupdate_scatter(hist, [lanes * _HSTRIDE + digit], ones,
                                           mask=jnp.full((16,), True))
                else:
                    pfx = jax.lax.shift_right_logical(bits, pshift)
                    plsc.addupdate_scatter(
                        hist, [lanes * _HSTRIDE + off0 + digit], ones,
                        mask=pfx == pre_b)
                    plsc.addupdate_scatter(
                        hist, [lanes * _HSTRIDE + off1 + digit], ones,
                        mask=pfx == pre_t)

        pltpu.async_copy(chunk_slice(0), buf.at[0], sems[0])
        pltpu.async_copy(chunk_slice(1), buf.at[1], sems[1])

        def chunk_body(j, _):
            pltpu.make_async_copy(chunk_slice(2 * j), buf.at[0], sems[0]).wait()
            process(0)

            @pl.when(j < _NCHUNK // 2 - 1)
            def _():
                pltpu.async_copy(chunk_slice(2 * j + 2), buf.at[0], sems[0])
            pltpu.make_async_copy(chunk_slice(2 * j + 1), buf.at[1], sems[1]).wait()
            process(1)

            @pl.when(j < _NCHUNK // 2 - 1)
            def _():
                pltpu.async_copy(chunk_slice(2 * j + 3), buf.at[1], sems[1])
            return 0
        jax.lax.fori_loop(0, _NCHUNK // 2, chunk_body, 0)

    def scan(shift, mask, pshift, pre_b, pre_t, off0, off1, first):
        @pl.when(cid == 0)
        def _():
            scan_ref(w1_ref, shift, mask, pshift, pre_b, pre_t, off0, off1, first)

        @pl.when(cid == 1)
        def _():
            scan_ref(w2_ref, shift, mask, pshift, pre_b, pre_t, off0, off1, first)

    def fold_publish(nb, off0, off1):
        for rank, off in ((0, off0), (1, off1)):
            @plsc.parallel_loop(0, nb // 16, 1, unroll=4)
            def _(g, off=off, rank=rank):
                acc = zeros16
                for lane in range(16):
                    acc = acc + hist[pl.ds(lane * _HSTRIDE + off + g * 16, 16)]
                pub[pl.ds(rank * 2048 + g * 16, 16)] = acc
            if nb < 2048:
                @plsc.parallel_loop(0, (2048 - nb) // 16, 1, unroll=4)
                def _(g, rank=rank):
                    pub[pl.ds(rank * 2048 + nb + g * 16, 16)] = zeros16
        pltpu.sync_copy(pub, spub.at[sid])

    def merge_find(states, nbits):
        w = 128
        plsc.subcore_barrier()
        for rank, stot in ((0, stot0), (1, stot1)):
            col = pl.multiple_of(rank * 2048 + sid * w, 128)
            pltpu.sync_copy(spub.at[:, pl.ds(col, w)], tmp)
            s_tot = jnp.int32(0)
            for g in range(w // 16):
                m = zeros16
                for row in range(16):
                    m = m + tmp[row, pl.ds(g * 16, 16)]
                cs = plsc.cumsum(m) + s_tot
                mraw[rank, pl.ds(g * 16, 16)] = m
                mcs[rank, pl.ds(g * 16, 16)] = cs
                s_tot = s_tot + jnp.sum(m)
            small[...] = jnp.full((16,), s_tot, jnp.int32)
            pltpu.sync_copy(small, stot.at[sid, pl.ds(0, 16)])
        plsc.subcore_barrier()
        for rank, stot in ((0, stot0), (1, stot1)):
            prefix, resid = states[rank]
            pltpu.sync_copy(stot.at[:, pl.ds(0, 16)], tmp16)
            tvec = zeros16
            for j in range(16):
                tvec = jnp.where(lanes == j, tmp16[j], tvec)
            ct = plsc.cumsum(tvec)
            gexcl = ct - tvec
            my_before = jnp.sum(jnp.where(lanes == sid, gexcl, 0))
            s_me = jnp.sum(jnp.where(lanes == sid, tvec, 0))
            is_owner = (my_before < resid) & (my_before + s_me >= resid)
            found = jnp.int32(0)
            bucket = jnp.int32(0)
            cbef = jnp.int32(0)
            for g in range(w // 16):
                cs = mcs[rank, pl.ds(g * 16, 16)]
                m = mraw[rank, pl.ds(g * 16, 16)]
                hit = (my_before + cs) >= resid
                pc = jnp.max(plsc.all_reduce_population_count(hit))
                lane_s = jnp.max(plsc.all_reduce_ffs(hit))
                csl = jnp.sum(jnp.where(lanes == lane_s, cs, 0))
                ml = jnp.sum(jnp.where(lanes == lane_s, m, 0))
                take = (found == 0) & (pc > 0)
                found = jnp.where(take, jnp.int32(1), found)
                bucket = jnp.where(take, sid * w + g * 16 + lane_s, bucket)
                cbef = jnp.where(take, my_before + csl - ml, cbef)
            new_prefix = jax.lax.shift_left(prefix, nbits) | bucket
            new_resid = resid - cbef

            @pl.when(is_owner)
            def _():
                small[...] = jnp.full((16,), new_prefix, jnp.int32)
                pltpu.sync_copy(small, sresp.at[rank, pl.ds(0, 16)])
                small[...] = jnp.full((16,), new_resid, jnp.int32)
                pltpu.sync_copy(small, sresr.at[rank, pl.ds(0, 16)])
        plsc.subcore_barrier()
        out = []
        for rank in range(2):
            pltpu.sync_copy(sresp.at[rank, pl.ds(0, 16)], rr.at[0])
            pltpu.sync_copy(sresr.at[rank, pl.ds(0, 16)], rr.at[1])
            out.append((jnp.max(rr[0]), jnp.max(rr[1])))
        return out

    r_bot = jnp.int32(_KW)
    r_top = jnp.int32(_NW - _KW + 1)
    zero = jnp.int32(0)

    scan(20, 2047, 0, zero, zero, 0, 0, True)
    fold_publish(2048, 0, 0)
    st1 = merge_find(((zero, r_bot), (zero, r_top)), 11)

    scan(9, 2047, 20, st1[0][0], st1[1][0], 0, 2048, False)
    fold_publish(2048, 0, 2048)
    st2 = merge_find(st1, 11)

    scan(0, 511, 9, st2[0][0], st2[1][0], 0, 512, False)
    fold_publish(512, 0, 512)
    st3 = merge_find(st2, 9)

    @pl.when(sid == 0)
    def _():
        ov = jnp.where(lanes == 0, jnp.full((16,), st3[0][0], jnp.int32), zeros16)
        ov = jnp.where(lanes == 1, jnp.full((16,), st3[1][0], jnp.int32), ov)
        small[...] = ov
        pltpu.sync_copy(small, out_ref.at[cid])


def _sc_select(w1_flat, w2_flat):
    mesh = plsc.VectorSubcoreMesh(core_axis_name="c", subcore_axis_name="s")
    return pl.kernel(
        _sc_select_body,
        out_type=jax.ShapeDtypeStruct((2, 16), jnp.int32),
        mesh=mesh,
        compiler_params=pltpu.CompilerParams(
            needs_layout_passes=False, use_tc_tiling_on_sc=False),
        scratch_types=[
            pltpu.VMEM((2, _CHUNK), jnp.float32),
            pltpu.VMEM((65536,), jnp.int32),
            pltpu.VMEM((4096,), jnp.int32),
            pltpu.VMEM((16, 128), jnp.int32),
            pltpu.VMEM((16, 16), jnp.int32),
            pltpu.VMEM((2, 128), jnp.int32),
            pltpu.VMEM((2, 128), jnp.int32),
            pltpu.VMEM((16,), jnp.int32),
            pltpu.VMEM((2, 16), jnp.int32),
            pltpu.VMEM_SHARED((16, 4096), jnp.int32),
            pltpu.VMEM_SHARED((16, 128), jnp.int32),
            pltpu.VMEM_SHARED((16, 128), jnp.int32),
            pltpu.VMEM_SHARED((2, 128), jnp.int32),
            pltpu.VMEM_SHARED((2, 128), jnp.int32),
            pltpu.SemaphoreType.DMA,
            pltpu.SemaphoreType.DMA,
        ],
    )(w1_flat, w2_flat)


def _select2(w_ref, r_bot, r_top):
    def bits():
        return jax.lax.bitcast_convert_type(w_ref[...], jnp.int32) & _ABS_MASK

    def body(_, carry):
        lo_b, hi_b, lo_t, hi_t = carry
        mid_b = (lo_b + hi_b) >> 1
        mid_t = (lo_t + hi_t) >> 1
        b = bits()
        c_b = jnp.sum((b <= mid_b).astype(jnp.int32))
        c_t = jnp.sum((b <= mid_t).astype(jnp.int32))
        ge_b = c_b >= r_bot
        ge_t = c_t >= r_top
        return (
            jnp.where(ge_b, lo_b, mid_b),
            jnp.where(ge_b, mid_b, hi_b),
            jnp.where(ge_t, lo_t, mid_t),
            jnp.where(ge_t, mid_t, hi_t),
        )

    init = (jnp.int32(-1), jnp.int32(0x7FFFFFFF), jnp.int32(-1), jnp.int32(0x7FFFFFFF))
    _, hi_b, _, hi_t = jax.lax.fori_loop(0, 31, body, init)
    return hi_b, hi_t


def _apply_pert(w_ref, hi_b, hi_t, out_dtype):
    b = jax.lax.bitcast_convert_type(w_ref[...], jnp.int32) & _ABS_MASK
    bot = (b <= hi_b).astype(jnp.float32)
    top = (b >= hi_t).astype(jnp.float32)
    return (w_ref[...] * (1.0 + _SCALE * (bot - top))).astype(out_dtype)


def _pert_body(w1_ref, b1_ref, w2_ref, b2_ref, thr_ref,
               pw1_ref, pb1_ref, pw2_ref, pb2_ref):
    for b_ref, p_ref, n in ((b1_ref, pb1_ref, _DF), (b2_ref, pb2_ref, _DM)):
        k = max(1, int(n * _SPARSITY / 2))
        hi_b, hi_t = _select2(b_ref, jnp.int32(k), jnp.int32(n - k + 1))
        p_ref[...] = _apply_pert(b_ref, hi_b, hi_t, jnp.float32)
    for i, (w_ref, p_ref) in enumerate(((w1_ref, pw1_ref), (w2_ref, pw2_ref))):
        p_ref[...] = _apply_pert(w_ref, thr_ref[i, 0], thr_ref[i, 1], jnp.bfloat16)


def _clean_body(
    x_ref, g_ref, bt_ref, wr_ref, br_ref,
    w1_ref, b1_ref, w2_ref, b2_ref,
    oa_ref, p8_ref,
):
    xb = x_ref[...]
    m = jnp.mean(xb, axis=-1, keepdims=True)
    xc = xb - m
    v = jnp.mean(xc * xc, axis=-1, keepdims=True)
    h = xc * jax.lax.rsqrt(v + 1e-5) * g_ref[...] + bt_ref[...]
    logits = (
        jnp.dot(h.astype(jnp.bfloat16), wr_ref[...].astype(jnp.bfloat16),
                preferred_element_type=jnp.float32)
        + br_ref[...]
    )
    mx = jnp.max(logits, axis=-1, keepdims=True)
    e = jnp.exp(logits - mx)
    p0 = e[:, :1] / jnp.sum(e, axis=-1, keepdims=True)
    hc = jax.nn.gelu(
        jnp.dot(xb.astype(jnp.bfloat16), w1_ref[...],
                preferred_element_type=jnp.float32) + b1_ref[...]
    )
    oa_ref[...] = (
        jnp.dot((p0 * hc).astype(jnp.bfloat16), w2_ref[...],
                preferred_element_type=jnp.float32)
        + p0 * b2_ref[...]
    )
    p8_ref[...] = jnp.broadcast_to(p0, p0.shape[:1] + (_NE,))


def _pert_mlp_body(
    x_ref, p8_ref, oa_ref, pw1_ref, pb1_ref, pw2_ref, pb2_ref, o_ref,
):
    pr = 1.0 - p8_ref[:, :1]
    hp = jax.nn.gelu(
        jnp.dot(x_ref[...].astype(jnp.bfloat16), pw1_ref[...],
                preferred_element_type=jnp.float32) + pb1_ref[...]
    )
    o_ref[...] = (
        oa_ref[...]
        + jnp.dot((pr * hp).astype(jnp.bfloat16), pw2_ref[...],
                  preferred_element_type=jnp.float32)
        + pr * pb2_ref[...]
    )


def kernel(x, ln_g, ln_b, Wr, br, W1, b1, W2, b2):
    thr = _sc_select(W1.reshape(-1), W2.reshape(-1))

    b1r = b1.reshape(1, _DF)
    b2r = b2.reshape(1, _DM)
    full = lambda shape: pl.BlockSpec(shape, lambda i: (0, 0))
    tok = lambda cols: pl.BlockSpec((_TB, cols), lambda i: (i, 0))

    out_a, p8 = pl.pallas_call(
        _clean_body,
        grid=(_T // _TB,),
        in_specs=[
            tok(_DM),
            full((1, _DM)),
            full((1, _DM)),
            full((_DM, _NE)),
            full((1, _NE)),
            full((_DM, _DF)),
            full((1, _DF)),
            full((_DF, _DM)),
            full((1, _DM)),
        ],
        out_specs=[tok(_DM), tok(_NE)],
        out_shape=[
            jax.ShapeDtypeStruct((_T, _DM), jnp.float32),
            jax.ShapeDtypeStruct((_T, _NE), jnp.float32),
        ],
    )(
        x, ln_g.reshape(1, _DM), ln_b.reshape(1, _DM), Wr, br.reshape(1, _NE),
        W1.astype(jnp.bfloat16), b1r, W2.astype(jnp.bfloat16), b2r,
    )

    pW1, pb1, pW2, pb2 = pl.pallas_call(
        _pert_body,
        out_shape=[
            jax.ShapeDtypeStruct((_DM, _DF), jnp.bfloat16),
            jax.ShapeDtypeStruct((1, _DF), jnp.float32),
            jax.ShapeDtypeStruct((_DF, _DM), jnp.bfloat16),
            jax.ShapeDtypeStruct((1, _DM), jnp.float32),
        ],
    )(W1, b1r, W2, b2r, thr)

    out = pl.pallas_call(
        _pert_mlp_body,
        grid=(_T // _TB,),
        in_specs=[
            tok(_DM),
            tok(_NE),
            tok(_DM),
            full((_DM, _DF)),
            full((1, _DF)),
            full((_DF, _DM)),
            full((1, _DM)),
        ],
        out_specs=tok(_DM),
        out_shape=jax.ShapeDtypeStruct((_T, _DM), jnp.float32),
    )(x, p8, out_a, pW1, pb1, pW2, pb2)
    return out


# speedup vs baseline: 1.5902x; 1.5902x over previous
import functools

import jax
import jax.numpy as jnp
from jax.experimental import pallas as pl
from jax.experimental.pallas import tpu as pltpu
from jax.experimental.pallas import tpu_sc as plsc

_NE = 8
_DM = 768
_DF = 3072
_SCALE = 0.8
_SPARSITY = 0.1
_T = 4096
_TB = 256

_ABS_MASK = 0x7FFFFFFF
_NW = _DM * _DF
_KW = max(1, int(_NW * _SPARSITY / 2))
_NTILE = 16
_CHUNK = 8192
_PER_TILE = _NW // _NTILE
_NCHUNK = _PER_TILE // _CHUNK
_HSTRIDE = 4096


def _sc_select_body(w1_ref, w2_ref, out_ref,
                    buf, hist, pub, tmp, tmp16, mcs, mraw, small, rr,
                    spub, stot0, stot1, sresp, sresr, sem0, sem1):
    cid = jax.lax.axis_index("c")
    sid = jax.lax.axis_index("s")
    lanes = jax.lax.broadcasted_iota(jnp.int32, (16,), 0)
    ones = jnp.ones((16,), jnp.int32)
    zeros16 = jnp.zeros((16,), jnp.int32)
    sems = (sem0, sem1)

    def scan_ref(w_ref, shift, mask, pshift, pre_b, pre_t, off0, off1, first):
        @plsc.parallel_loop(0, 65536 // 16, 1, unroll=8)
        def _(i):
            hist[pl.ds(i * 16, 16)] = zeros16
        base = sid * _PER_TILE

        def chunk_slice(k):
            return w_ref.at[pl.ds(base + k * _CHUNK, _CHUNK)]

        def process(cur):
            @plsc.parallel_loop(0, _CHUNK // 16, 1, unroll=8)
            def _(g):
                v = buf[cur, pl.ds(g * 16, 16)]
                bits = jax.lax.bitcast_convert_type(v, jnp.int32) & _ABS_MASK
                digit = jax.lax.shift_right_logical(bits, shift) & mask
                if first:
                    plsc.addupdate_scatter(hist, [lanes * _HSTRIDE + digit], ones,
                                           mask=jnp.full((16,), True))
                else:
                    pfx = jax.lax.shift_right_logical(bits, pshift)
                    plsc.addupdate_scatter(
                        hist, [lanes * _HSTRIDE + off0 + digit], ones,
                        mask=pfx == pre_b)
                    plsc.addupdate_scatter(
                        hist, [lanes * _HSTRIDE + off1 + digit], ones,
                        mask=pfx == pre_t)

        pltpu.async_copy(chunk_slice(0), buf.at[0], sems[0])
        pltpu.async_copy(chunk_slice(1), buf.at[1], sems[1])

        def chunk_body(j, _):
            pltpu.make_async_copy(chunk_slice(2 * j), buf.at[0], sems[0]).wait()
            process(0)

            @pl.when(j < _NCHUNK // 2 - 1)
            def _():
                pltpu.async_copy(chunk_slice(2 * j + 2), buf.at[0], sems[0])
            pltpu.make_async_copy(chunk_slice(2 * j + 1), buf.at[1], sems[1]).wait()
            process(1)

            @pl.when(j < _NCHUNK // 2 - 1)
            def _():
                pltpu.async_copy(chunk_slice(2 * j + 3), buf.at[1], sems[1])
            return 0
        jax.lax.fori_loop(0, _NCHUNK // 2, chunk_body, 0)

    def scan(shift, mask, pshift, pre_b, pre_t, off0, off1, first):
        @pl.when(cid == 0)
        def _():
            scan_ref(w1_ref, shift, mask, pshift, pre_b, pre_t, off0, off1, first)

        @pl.when(cid == 1)
        def _():
            scan_ref(w2_ref, shift, mask, pshift, pre_b, pre_t, off0, off1, first)

    def fold_publish(nb, off0, off1):
        for rank, off in ((0, off0), (1, off1)):
            @plsc.parallel_loop(0, nb // 16, 1, unroll=4)
            def _(g, off=off, rank=rank):
                acc = zeros16
                for lane in range(16):
                    acc = acc + hist[pl.ds(lane * _HSTRIDE + off + g * 16, 16)]
                pub[pl.ds(rank * 2048 + g * 16, 16)] = acc
            if nb < 2048:
                @plsc.parallel_loop(0, (2048 - nb) // 16, 1, unroll=4)
                def _(g, rank=rank):
                    pub[pl.ds(rank * 2048 + nb + g * 16, 16)] = zeros16
        pltpu.sync_copy(pub, spub.at[sid])

    def merge_find(states, nbits):
        w = 128
        plsc.subcore_barrier()
        for rank, stot in ((0, stot0), (1, stot1)):
            col = pl.multiple_of(rank * 2048 + sid * w, 128)
            pltpu.sync_copy(spub.at[:, pl.ds(col, w)], tmp)
            s_tot = jnp.int32(0)
            for g in range(w // 16):
                m = zeros16
                for row in range(16):
                    m = m + tmp[row, pl.ds(g * 16, 16)]
                cs = plsc.cumsum(m) + s_tot
                mraw[rank, pl.ds(g * 16, 16)] = m
                mcs[rank, pl.ds(g * 16, 16)] = cs
                s_tot = s_tot + jnp.sum(m)
            small[...] = jnp.full((16,), s_tot, jnp.int32)
            pltpu.sync_copy(small, stot.at[sid, pl.ds(0, 16)])
        plsc.subcore_barrier()
        for rank, stot in ((0, stot0), (1, stot1)):
            prefix, resid = states[rank]
            pltpu.sync_copy(stot.at[:, pl.ds(0, 16)], tmp16)
            tvec = zeros16
            for j in range(16):
                tvec = jnp.where(lanes == j, tmp16[j], tvec)
            ct = plsc.cumsum(tvec)
            gexcl = ct - tvec
            my_before = jnp.sum(jnp.where(lanes == sid, gexcl, 0))
            s_me = jnp.sum(jnp.where(lanes == sid, tvec, 0))
            is_owner = (my_before < resid) & (my_before + s_me >= resid)
            found = jnp.int32(0)
            bucket = jnp.int32(0)
            cbef = jnp.int32(0)
            for g in range(w // 16):
                cs = mcs[rank, pl.ds(g * 16, 16)]
                m = mraw[rank, pl.ds(g * 16, 16)]
                hit = (my_before + cs) >= resid
                pc = jnp.max(plsc.all_reduce_population_count(hit))
                lane_s = jnp.max(plsc.all_reduce_ffs(hit))
                csl = jnp.sum(jnp.where(lanes == lane_s, cs, 0))
                ml = jnp.sum(jnp.where(lanes == lane_s, m, 0))
                take = (found == 0) & (pc > 0)
                found = jnp.where(take, jnp.int32(1), found)
                bucket = jnp.where(take, sid * w + g * 16 + lane_s, bucket)
                cbef = jnp.where(take, my_before + csl - ml, cbef)
            new_prefix = jax.lax.shift_left(prefix, nbits) | bucket
            new_resid = resid - cbef

            @pl.when(is_owner)
            def _():
                small[...] = jnp.full((16,), new_prefix, jnp.int32)
                pltpu.sync_copy(small, sresp.at[rank, pl.ds(0, 16)])
                small[...] = jnp.full((16,), new_resid, jnp.int32)
                pltpu.sync_copy(small, sresr.at[rank, pl.ds(0, 16)])
        plsc.subcore_barrier()
        out = []
        for rank in range(2):
            pltpu.sync_copy(sresp.at[rank, pl.ds(0, 16)], rr.at[0])
            pltpu.sync_copy(sresr.at[rank, pl.ds(0, 16)], rr.at[1])
            out.append((jnp.max(rr[0]), jnp.max(rr[1])))
        return out

    r_bot = jnp.int32(_KW)
    r_top = jnp.int32(_NW - _KW + 1)
    zero = jnp.int32(0)

    scan(20, 2047, 0, zero, zero, 0, 0, True)
    fold_publish(2048, 0, 0)
    st1 = merge_find(((zero, r_bot), (zero, r_top)), 11)

    scan(9, 2047, 20, st1[0][0], st1[1][0], 0, 2048, False)
    fold_publish(2048, 0, 2048)
    st2 = merge_find(st1, 11)

    scan(0, 511, 9, st2[0][0], st2[1][0], 0, 512, False)
    fold_publish(512, 0, 512)
    st3 = merge_find(st2, 9)

    @pl.when(sid == 0)
    def _():
        ov = jnp.where(lanes == 0, jnp.full((16,), st3[0][0], jnp.int32), zeros16)
        ov = jnp.where(lanes == 1, jnp.full((16,), st3[1][0], jnp.int32), ov)
        small[...] = ov
        pltpu.sync_copy(small, out_ref.at[cid])


def _sc_select(w1_flat, w2_flat):
    mesh = plsc.VectorSubcoreMesh(core_axis_name="c", subcore_axis_name="s")
    return pl.kernel(
        _sc_select_body,
        out_type=jax.ShapeDtypeStruct((2, 16), jnp.int32),
        mesh=mesh,
        compiler_params=pltpu.CompilerParams(
            needs_layout_passes=False, use_tc_tiling_on_sc=False),
        scratch_types=[
            pltpu.VMEM((2, _CHUNK), jnp.float32),
            pltpu.VMEM((65536,), jnp.int32),
            pltpu.VMEM((4096,), jnp.int32),
            pltpu.VMEM((16, 128), jnp.int32),
            pltpu.VMEM((16, 16), jnp.int32),
            pltpu.VMEM((2, 128), jnp.int32),
            pltpu.VMEM((2, 128), jnp.int32),
            pltpu.VMEM((16,), jnp.int32),
            pltpu.VMEM((2, 16), jnp.int32),
            pltpu.VMEM_SHARED((16, 4096), jnp.int32),
            pltpu.VMEM_SHARED((16, 128), jnp.int32),
            pltpu.VMEM_SHARED((16, 128), jnp.int32),
            pltpu.VMEM_SHARED((2, 128), jnp.int32),
            pltpu.VMEM_SHARED((2, 128), jnp.int32),
            pltpu.SemaphoreType.DMA,
            pltpu.SemaphoreType.DMA,
        ],
    )(w1_flat, w2_flat)


def _select2(w_ref, r_bot, r_top):
    def bits():
        return jax.lax.bitcast_convert_type(w_ref[...], jnp.int32) & _ABS_MASK

    def body(_, carry):
        lo_b, hi_b, lo_t, hi_t = carry
        mid_b = (lo_b + hi_b) >> 1
        mid_t = (lo_t + hi_t) >> 1
        b = bits()
        c_b = jnp.sum((b <= mid_b).astype(jnp.int32))
        c_t = jnp.sum((b <= mid_t).astype(jnp.int32))
        ge_b = c_b >= r_bot
        ge_t = c_t >= r_top
        return (
            jnp.where(ge_b, lo_b, mid_b),
            jnp.where(ge_b, mid_b, hi_b),
            jnp.where(ge_t, lo_t, mid_t),
            jnp.where(ge_t, mid_t, hi_t),
        )

    init = (jnp.int32(-1), jnp.int32(0x7FFFFFFF), jnp.int32(-1), jnp.int32(0x7FFFFFFF))
    _, hi_b, _, hi_t = jax.lax.fori_loop(0, 31, body, init)
    return hi_b, hi_t


def _apply_pert(w_ref, hi_b, hi_t, out_dtype):
    b = jax.lax.bitcast_convert_type(w_ref[...], jnp.int32) & _ABS_MASK
    bot = (b <= hi_b).astype(jnp.float32)
    top = (b >= hi_t).astype(jnp.float32)
    return (w_ref[...] * (1.0 + _SCALE * (bot - top))).astype(out_dtype)


def _pert_body(w1_ref, b1_ref, w2_ref, b2_ref, thr_ref,
               pw1_ref, pb1_ref, pw2_ref, pb2_ref):
    for b_ref, p_ref, n in ((b1_ref, pb1_ref, _DF), (b2_ref, pb2_ref, _DM)):
        k = max(1, int(n * _SPARSITY / 2))
        hi_b, hi_t = _select2(b_ref, jnp.int32(k), jnp.int32(n - k + 1))
        p_ref[...] = _apply_pert(b_ref, hi_b, hi_t, jnp.float32)
    for i, (w_ref, p_ref) in enumerate(((w1_ref, pw1_ref), (w2_ref, pw2_ref))):
        p_ref[...] = _apply_pert(w_ref, thr_ref[i, 0], thr_ref[i, 1], jnp.bfloat16)


def _clean_body(
    x_ref, g_ref, bt_ref, wr_ref, br_ref,
    w1_ref, b1_ref, w2_ref, b2_ref,
    oa_ref, p8_ref,
):
    xb = x_ref[...]
    m = jnp.mean(xb, axis=-1, keepdims=True)
    xc = xb - m
    v = jnp.mean(xc * xc, axis=-1, keepdims=True)
    h = xc * jax.lax.rsqrt(v + 1e-5) * g_ref[...] + bt_ref[...]
    logits = (
        jnp.dot(h.astype(jnp.bfloat16), wr_ref[...].astype(jnp.bfloat16),
                preferred_element_type=jnp.float32)
        + br_ref[...]
    )
    mx = jnp.max(logits, axis=-1, keepdims=True)
    e = jnp.exp(logits - mx)
    p0 = e[:, :1] / jnp.sum(e, axis=-1, keepdims=True)
    hc = jax.nn.gelu(
        jnp.dot(xb.astype(jnp.bfloat16), w1_ref[...],
                preferred_element_type=jnp.float32) + b1_ref[...]
    )
    oa_ref[...] = (
        jnp.dot((p0 * hc).astype(jnp.bfloat16), w2_ref[...],
                preferred_element_type=jnp.float32)
        + p0 * b2_ref[...]
    )
    p8_ref[...] = jnp.broadcast_to(p0, p0.shape[:1] + (_NE,))


def _pert_mlp_body(
    x_ref, p8_ref, oa_ref, pw1_ref, pb1_ref, pw2_ref, pb2_ref, o_ref,
):
    pr = 1.0 - p8_ref[:, :1]
    hp = jax.nn.gelu(
        jnp.dot(x_ref[...].astype(jnp.bfloat16), pw1_ref[...],
                preferred_element_type=jnp.float32) + pb1_ref[...]
    )
    o_ref[...] = (
        oa_ref[...]
        + jnp.dot((pr * hp).astype(jnp.bfloat16), pw2_ref[...],
                  preferred_element_type=jnp.float32)
        + pr * pb2_ref[...]
    )


def kernel(x, ln_g, ln_b, Wr, br, W1, b1, W2, b2):
    thr = _sc_select(W1.reshape(-1), W2.reshape(-1))

    b1r = b1.reshape(1, _DF)
    b2r = b2.reshape(1, _DM)
    full = lambda shape: pl.BlockSpec(shape, lambda i: (0, 0))
    tok = lambda cols: pl.BlockSpec((_TB, cols), lambda i: (i, 0))

    out_a, p8 = pl.pallas_call(
        _clean_body,
        grid=(_T // _TB,),
        in_specs=[
            tok(_DM),
            full((1, _DM)),
            full((1, _DM)),
            full((_DM, _NE)),
            full((1, _NE)),
            full((_DM, _DF)),
            full((1, _DF)),
            full((_DF, _DM)),
            full((1, _DM)),
        ],
        out_specs=[tok(_DM), tok(_NE)],
        out_shape=[
            jax.ShapeDtypeStruct((_T, _DM), jnp.float32),
            jax.ShapeDtypeStruct((_T, _NE), jnp.float32),
        ],
    )(
        x, ln_g.reshape(1, _DM), ln_b.reshape(1, _DM), Wr, br.reshape(1, _NE),
        W1.astype(jnp.bfloat16), b1r, W2.astype(jnp.bfloat16), b2r,
    )

    pW1, pb1, pW2, pb2 = pl.pallas_call(
        _pert_body,
        out_shape=[
            jax.ShapeDtypeStruct((_DM, _DF), jnp.bfloat16),
            jax.ShapeDtypeStruct((1, _DF), jnp.float32),
            jax.ShapeDtypeStruct((_DF, _DM), jnp.bfloat16),
            jax.ShapeDtypeStruct((1, _DM), jnp.float32),
        ],
    )(W1, b1r, W2, b2r, thr)

    out = pl.pallas_call(
        _pert_mlp_body,
        grid=(_T // _TB,),
        in_specs=[
            tok(_DM),
            tok(_NE),
            tok(_DM),
            full((_DM, _DF)),
            full((1, _DF)),
            full((_DF, _DM)),
            full((1, _DM)),
        ],
        out_specs=tok(_DM),
        out_shape=jax.ShapeDtypeStruct((_T, _DM), jnp.float32),
    )(x, p8, out_a, pW1, pb1, pW2, pb2)
    return out


# speedup vs baseline: 1.6260x; 1.0226x over previous
import functools

import jax
import jax.numpy as jnp
from jax.experimental import pallas as pl
from jax.experimental.pallas import tpu as pltpu
from jax.experimental.pallas import tpu_sc as plsc

_NE = 8
_DM = 768
_DF = 3072
_SCALE = 0.8
_SPARSITY = 0.1
_T = 4096
_TB = 512

_ABS_MASK = 0x7FFFFFFF
_NW = _DM * _DF
_KW = max(1, int(_NW * _SPARSITY / 2))
_NTILE = 16
_CHUNK = 8192
_PER_TILE = _NW // _NTILE
_NCHUNK = _PER_TILE // _CHUNK
_HSTRIDE = 4096


def _sc_select_body(w1_ref, w2_ref, out_ref,
                    buf, hist, pub, tmp, tmp16, mcs, mraw, small, rr,
                    spub, stot0, stot1, sresp, sresr, sem0, sem1):
    cid = jax.lax.axis_index("c")
    sid = jax.lax.axis_index("s")
    lanes = jax.lax.broadcasted_iota(jnp.int32, (16,), 0)
    ones = jnp.ones((16,), jnp.int32)
    zeros16 = jnp.zeros((16,), jnp.int32)
    sems = (sem0, sem1)

    def scan_ref(w_ref, shift, mask, pshift, pre_b, pre_t, off0, off1, first):
        @plsc.parallel_loop(0, 65536 // 16, 1, unroll=8)
        def _(i):
            hist[pl.ds(i * 16, 16)] = zeros16
        base = sid * _PER_TILE

        def chunk_slice(k):
            return w_ref.at[pl.ds(base + k * _CHUNK, _CHUNK)]

        def process(cur):
            @plsc.parallel_loop(0, _CHUNK // 16, 1, unroll=8)
            def _(g):
                v = buf[cur, pl.ds(g * 16, 16)]
                bits = jax.lax.bitcast_convert_type(v, jnp.int32) & _ABS_MASK
                digit = jax.lax.shift_right_logical(bits, shift) & mask
                if first:
                    plsc.addupdate_scatter(hist, [lanes * _HSTRIDE + digit], ones,
                                           mask=jnp.full((16,), True))
                else:
                    pfx = jax.lax.shift_right_logical(bits, pshift)
                    plsc.addupdate_scatter(
                        hist, [lanes * _HSTRIDE + off0 + digit], ones,
                        mask=pfx == pre_b)
                    plsc.addupdate_scatter(
                        hist, [lanes * _HSTRIDE + off1 + digit], ones,
                        mask=pfx == pre_t)

        pltpu.async_copy(chunk_slice(0), buf.at[0], sems[0])
        pltpu.async_copy(chunk_slice(1), buf.at[1], sems[1])

        def chunk_body(j, _):
            pltpu.make_async_copy(chunk_slice(2 * j), buf.at[0], sems[0]).wait()
            process(0)

            @pl.when(j < _NCHUNK // 2 - 1)
            def _():
                pltpu.async_copy(chunk_slice(2 * j + 2), buf.at[0], sems[0])
            pltpu.make_async_copy(chunk_slice(2 * j + 1), buf.at[1], sems[1]).wait()
            process(1)

            @pl.when(j < _NCHUNK // 2 - 1)
            def _():
                pltpu.async_copy(chunk_slice(2 * j + 3), buf.at[1], sems[1])
            return 0
        jax.lax.fori_loop(0, _NCHUNK // 2, chunk_body, 0)

    def scan(shift, mask, pshift, pre_b, pre_t, off0, off1, first):
        @pl.when(cid == 0)
        def _():
            scan_ref(w1_ref, shift, mask, pshift, pre_b, pre_t, off0, off1, first)

        @pl.when(cid == 1)
        def _():
            scan_ref(w2_ref, shift, mask, pshift, pre_b, pre_t, off0, off1, first)

    def fold_publish(nb, off0, off1):
        for rank, off in ((0, off0), (1, off1)):
            @plsc.parallel_loop(0, nb // 16, 1, unroll=4)
            def _(g, off=off, rank=rank):
                acc = zeros16
                for lane in range(16):
                    acc = acc + hist[pl.ds(lane * _HSTRIDE + off + g * 16, 16)]
                pub[pl.ds(rank * 2048 + g * 16, 16)] = acc
            if nb < 2048:
                @plsc.parallel_loop(0, (2048 - nb) // 16, 1, unroll=4)
                def _(g, rank=rank):
                    pub[pl.ds(rank * 2048 + nb + g * 16, 16)] = zeros16
        pltpu.sync_copy(pub, spub.at[sid])

    def merge_find(states, nbits):
        w = 128
        plsc.subcore_barrier()
        for rank, stot in ((0, stot0), (1, stot1)):
            col = pl.multiple_of(rank * 2048 + sid * w, 128)
            pltpu.sync_copy(spub.at[:, pl.ds(col, w)], tmp)
            s_tot = jnp.int32(0)
            for g in range(w // 16):
                m = zeros16
                for row in range(16):
                    m = m + tmp[row, pl.ds(g * 16, 16)]
                cs = plsc.cumsum(m) + s_tot
                mraw[rank, pl.ds(g * 16, 16)] = m
                mcs[rank, pl.ds(g * 16, 16)] = cs
                s_tot = s_tot + jnp.sum(m)
            small[...] = jnp.full((16,), s_tot, jnp.int32)
            pltpu.sync_copy(small, stot.at[sid, pl.ds(0, 16)])
        plsc.subcore_barrier()
        for rank, stot in ((0, stot0), (1, stot1)):
            prefix, resid = states[rank]
            pltpu.sync_copy(stot.at[:, pl.ds(0, 16)], tmp16)
            tvec = zeros16
            for j in range(16):
                tvec = jnp.where(lanes == j, tmp16[j], tvec)
            ct = plsc.cumsum(tvec)
            gexcl = ct - tvec
            my_before = jnp.sum(jnp.where(lanes == sid, gexcl, 0))
            s_me = jnp.sum(jnp.where(lanes == sid, tvec, 0))
            is_owner = (my_before < resid) & (my_before + s_me >= resid)
            found = jnp.int32(0)
            bucket = jnp.int32(0)
            cbef = jnp.int32(0)
            for g in range(w // 16):
                cs = mcs[rank, pl.ds(g * 16, 16)]
                m = mraw[rank, pl.ds(g * 16, 16)]
                hit = (my_before + cs) >= resid
                pc = jnp.max(plsc.all_reduce_population_count(hit))
                lane_s = jnp.max(plsc.all_reduce_ffs(hit))
                csl = jnp.sum(jnp.where(lanes == lane_s, cs, 0))
                ml = jnp.sum(jnp.where(lanes == lane_s, m, 0))
                take = (found == 0) & (pc > 0)
                found = jnp.where(take, jnp.int32(1), found)
                bucket = jnp.where(take, sid * w + g * 16 + lane_s, bucket)
                cbef = jnp.where(take, my_before + csl - ml, cbef)
            new_prefix = jax.lax.shift_left(prefix, nbits) | bucket
            new_resid = resid - cbef

            @pl.when(is_owner)
            def _():
                small[...] = jnp.full((16,), new_prefix, jnp.int32)
                pltpu.sync_copy(small, sresp.at[rank, pl.ds(0, 16)])
                small[...] = jnp.full((16,), new_resid, jnp.int32)
                pltpu.sync_copy(small, sresr.at[rank, pl.ds(0, 16)])
        plsc.subcore_barrier()
        out = []
        for rank in range(2):
            pltpu.sync_copy(sresp.at[rank, pl.ds(0, 16)], rr.at[0])
            pltpu.sync_copy(sresr.at[rank, pl.ds(0, 16)], rr.at[1])
            out.append((jnp.max(rr[0]), jnp.max(rr[1])))
        return out

    r_bot = jnp.int32(_KW)
    r_top = jnp.int32(_NW - _KW + 1)
    zero = jnp.int32(0)

    scan(20, 2047, 0, zero, zero, 0, 0, True)
    fold_publish(2048, 0, 0)
    st1 = merge_find(((zero, r_bot), (zero, r_top)), 11)

    scan(9, 2047, 20, st1[0][0], st1[1][0], 0, 2048, False)
    fold_publish(2048, 0, 2048)
    st2 = merge_find(st1, 11)

    scan(0, 511, 9, st2[0][0], st2[1][0], 0, 512, False)
    fold_publish(512, 0, 512)
    st3 = merge_find(st2, 9)

    @pl.when(sid == 0)
    def _():
        ov = jnp.where(lanes == 0, jnp.full((16,), st3[0][0], jnp.int32), zeros16)
        ov = jnp.where(lanes == 1, jnp.full((16,), st3[1][0], jnp.int32), ov)
        small[...] = ov
        pltpu.sync_copy(small, out_ref.at[cid])


def _sc_select(w1_flat, w2_flat):
    mesh = plsc.VectorSubcoreMesh(core_axis_name="c", subcore_axis_name="s")
    return pl.kernel(
        _sc_select_body,
        out_type=jax.ShapeDtypeStruct((2, 16), jnp.int32),
        mesh=mesh,
        compiler_params=pltpu.CompilerParams(
            needs_layout_passes=False, use_tc_tiling_on_sc=False),
        scratch_types=[
            pltpu.VMEM((2, _CHUNK), jnp.float32),
            pltpu.VMEM((65536,), jnp.int32),
            pltpu.VMEM((4096,), jnp.int32),
            pltpu.VMEM((16, 128), jnp.int32),
            pltpu.VMEM((16, 16), jnp.int32),
            pltpu.VMEM((2, 128), jnp.int32),
            pltpu.VMEM((2, 128), jnp.int32),
            pltpu.VMEM((16,), jnp.int32),
            pltpu.VMEM((2, 16), jnp.int32),
            pltpu.VMEM_SHARED((16, 4096), jnp.int32),
            pltpu.VMEM_SHARED((16, 128), jnp.int32),
            pltpu.VMEM_SHARED((16, 128), jnp.int32),
            pltpu.VMEM_SHARED((2, 128), jnp.int32),
            pltpu.VMEM_SHARED((2, 128), jnp.int32),
            pltpu.SemaphoreType.DMA,
            pltpu.SemaphoreType.DMA,
        ],
    )(w1_flat, w2_flat)


def _select2(w_ref, r_bot, r_top):
    def bits():
        return jax.lax.bitcast_convert_type(w_ref[...], jnp.int32) & _ABS_MASK

    def body(_, carry):
        lo_b, hi_b, lo_t, hi_t = carry
        mid_b = (lo_b + hi_b) >> 1
        mid_t = (lo_t + hi_t) >> 1
        b = bits()
        c_b = jnp.sum((b <= mid_b).astype(jnp.int32))
        c_t = jnp.sum((b <= mid_t).astype(jnp.int32))
        ge_b = c_b >= r_bot
        ge_t = c_t >= r_top
        return (
            jnp.where(ge_b, lo_b, mid_b),
            jnp.where(ge_b, mid_b, hi_b),
            jnp.where(ge_t, lo_t, mid_t),
            jnp.where(ge_t, mid_t, hi_t),
        )

    init = (jnp.int32(-1), jnp.int32(0x7FFFFFFF), jnp.int32(-1), jnp.int32(0x7FFFFFFF))
    _, hi_b, _, hi_t = jax.lax.fori_loop(0, 31, body, init)
    return hi_b, hi_t


def _apply_pert(w_ref, hi_b, hi_t, out_dtype):
    b = jax.lax.bitcast_convert_type(w_ref[...], jnp.int32) & _ABS_MASK
    bot = (b <= hi_b).astype(jnp.float32)
    top = (b >= hi_t).astype(jnp.float32)
    return (w_ref[...] * (1.0 + _SCALE * (bot - top))).astype(out_dtype)


def _pert_body(w1_ref, b1_ref, w2_ref, b2_ref, thr_ref,
               pw1_ref, pb1_ref, pw2_ref, pb2_ref):
    for b_ref, p_ref, n in ((b1_ref, pb1_ref, _DF), (b2_ref, pb2_ref, _DM)):
        k = max(1, int(n * _SPARSITY / 2))
        hi_b, hi_t = _select2(b_ref, jnp.int32(k), jnp.int32(n - k + 1))
        p_ref[...] = _apply_pert(b_ref, hi_b, hi_t, jnp.float32)
    for i, (w_ref, p_ref) in enumerate(((w1_ref, pw1_ref), (w2_ref, pw2_ref))):
        p_ref[...] = _apply_pert(w_ref, thr_ref[i, 0], thr_ref[i, 1], jnp.bfloat16)


def _clean_body(
    x_ref, g_ref, bt_ref, wr_ref, br_ref,
    w1_ref, b1_ref, w2_ref, b2_ref,
    oa_ref, p8_ref,
):
    xb = x_ref[...]
    m = jnp.mean(xb, axis=-1, keepdims=True)
    xc = xb - m
    v = jnp.mean(xc * xc, axis=-1, keepdims=True)
    h = xc * jax.lax.rsqrt(v + 1e-5) * g_ref[...] + bt_ref[...]
    logits = (
        jnp.dot(h.astype(jnp.bfloat16), wr_ref[...].astype(jnp.bfloat16),
                preferred_element_type=jnp.float32)
        + br_ref[...]
    )
    mx = jnp.max(logits, axis=-1, keepdims=True)
    e = jnp.exp(logits - mx)
    p0 = e[:, :1] / jnp.sum(e, axis=-1, keepdims=True)
    hc = jax.nn.gelu(
        jnp.dot(xb.astype(jnp.bfloat16), w1_ref[...],
                preferred_element_type=jnp.float32) + b1_ref[...]
    )
    oa_ref[...] = (
        jnp.dot((p0 * hc).astype(jnp.bfloat16), w2_ref[...],
                preferred_element_type=jnp.float32)
        + p0 * b2_ref[...]
    )
    p8_ref[...] = jnp.broadcast_to(p0, p0.shape[:1] + (_NE,))


def _pert_mlp_body(
    x_ref, p8_ref, oa_ref, pw1_ref, pb1_ref, pw2_ref, pb2_ref, o_ref,
):
    pr = 1.0 - p8_ref[:, :1]
    hp = jax.nn.gelu(
        jnp.dot(x_ref[...].astype(jnp.bfloat16), pw1_ref[...],
                preferred_element_type=jnp.float32) + pb1_ref[...]
    )
    o_ref[...] = (
        oa_ref[...]
        + jnp.dot((pr * hp).astype(jnp.bfloat16), pw2_ref[...],
                  preferred_element_type=jnp.float32)
        + pr * pb2_ref[...]
    )


def kernel(x, ln_g, ln_b, Wr, br, W1, b1, W2, b2):
    thr = _sc_select(W1.reshape(-1), W2.reshape(-1))

    b1r = b1.reshape(1, _DF)
    b2r = b2.reshape(1, _DM)
    full = lambda shape: pl.BlockSpec(shape, lambda i: (0, 0))
    tok = lambda cols: pl.BlockSpec((_TB, cols), lambda i: (i, 0))

    out_a, p8 = pl.pallas_call(
        _clean_body,
        grid=(_T // _TB,),
        in_specs=[
            tok(_DM),
            full((1, _DM)),
            full((1, _DM)),
            full((_DM, _NE)),
            full((1, _NE)),
            full((_DM, _DF)),
            full((1, _DF)),
            full((_DF, _DM)),
            full((1, _DM)),
        ],
        out_specs=[tok(_DM), tok(_NE)],
        out_shape=[
            jax.ShapeDtypeStruct((_T, _DM), jnp.float32),
            jax.ShapeDtypeStruct((_T, _NE), jnp.float32),
        ],
    )(
        x, ln_g.reshape(1, _DM), ln_b.reshape(1, _DM), Wr, br.reshape(1, _NE),
        W1.astype(jnp.bfloat16), b1r, W2.astype(jnp.bfloat16), b2r,
    )

    pW1, pb1, pW2, pb2 = pl.pallas_call(
        _pert_body,
        out_shape=[
            jax.ShapeDtypeStruct((_DM, _DF), jnp.bfloat16),
            jax.ShapeDtypeStruct((1, _DF), jnp.float32),
            jax.ShapeDtypeStruct((_DF, _DM), jnp.bfloat16),
            jax.ShapeDtypeStruct((1, _DM), jnp.float32),
        ],
    )(W1, b1r, W2, b2r, thr)

    out = pl.pallas_call(
        _pert_mlp_body,
        grid=(_T // _TB,),
        in_specs=[
            tok(_DM),
            tok(_NE),
            tok(_DM),
            full((_DM, _DF)),
            full((1, _DF)),
            full((_DF, _DM)),
            full((1, _DM)),
        ],
        out_specs=tok(_DM),
        out_shape=jax.ShapeDtypeStruct((_T, _DM), jnp.float32),
    )(x, p8, out_a, pW1, pb1, pW2, pb2)
    return out


# speedup vs baseline: 1.6569x; 1.0190x over previous
import functools

import jax
import jax.numpy as jnp
from jax.experimental import pallas as pl
from jax.experimental.pallas import tpu as pltpu
from jax.experimental.pallas import tpu_sc as plsc

_NE = 8
_DM = 768
_DF = 3072
_SCALE = 0.8
_SPARSITY = 0.1
_T = 4096
_TB = 512

_ABS_MASK = 0x7FFFFFFF
_NW = _DM * _DF
_KW = max(1, int(_NW * _SPARSITY / 2))
_NTILE = 16
_CHUNK = 8192
_PER_TILE = _NW // _NTILE
_NCHUNK = _PER_TILE // _CHUNK
_HSTRIDE = 4096


def _sc_select_body(w1_ref, w2_ref, out_ref,
                    buf, hist, pub, tmp, tmp16, mcs, mraw, small, rr,
                    spub, stot0, stot1, sresp, sresr, sem0, sem1):
    cid = jax.lax.axis_index("c")
    sid = jax.lax.axis_index("s")
    lanes = jax.lax.broadcasted_iota(jnp.int32, (16,), 0)
    ones = jnp.ones((16,), jnp.int32)
    zeros16 = jnp.zeros((16,), jnp.int32)
    sems = (sem0, sem1)

    def scan_ref(w_ref, shift, mask, pshift, pre_b, pre_t, off0, off1, first, zgl):
        @plsc.parallel_loop(0, 16 << zgl, 1, unroll=8)
        def _(g):
            lane = jax.lax.shift_right_logical(g, zgl)
            within = (g & ((1 << zgl) - 1)) * 16
            hist[pl.ds(lane * _HSTRIDE + within, 16)] = zeros16
        base = sid * _PER_TILE

        def chunk_slice(k):
            return w_ref.at[pl.ds(base + k * _CHUNK, _CHUNK)]

        def process(cur):
            @plsc.parallel_loop(0, _CHUNK // 16, 1, unroll=8)
            def _(g):
                v = buf[cur, pl.ds(g * 16, 16)]
                bits = jax.lax.bitcast_convert_type(v, jnp.int32) & _ABS_MASK
                digit = jax.lax.shift_right_logical(bits, shift) & mask
                if first:
                    plsc.addupdate_scatter(hist, [lanes * _HSTRIDE + digit], ones,
                                           mask=jnp.full((16,), True))
                else:
                    pfx = jax.lax.shift_right_logical(bits, pshift)
                    plsc.addupdate_scatter(
                        hist, [lanes * _HSTRIDE + off0 + digit], ones,
                        mask=pfx == pre_b)
                    plsc.addupdate_scatter(
                        hist, [lanes * _HSTRIDE + off1 + digit], ones,
                        mask=pfx == pre_t)

        pltpu.async_copy(chunk_slice(0), buf.at[0], sems[0])
        pltpu.async_copy(chunk_slice(1), buf.at[1], sems[1])

        def chunk_body(j, _):
            pltpu.make_async_copy(chunk_slice(2 * j), buf.at[0], sems[0]).wait()
            process(0)

            @pl.when(j < _NCHUNK // 2 - 1)
            def _():
                pltpu.async_copy(chunk_slice(2 * j + 2), buf.at[0], sems[0])
            pltpu.make_async_copy(chunk_slice(2 * j + 1), buf.at[1], sems[1]).wait()
            process(1)

            @pl.when(j < _NCHUNK // 2 - 1)
            def _():
                pltpu.async_copy(chunk_slice(2 * j + 3), buf.at[1], sems[1])
            return 0
        jax.lax.fori_loop(0, _NCHUNK // 2, chunk_body, 0)

    def scan(shift, mask, pshift, pre_b, pre_t, off0, off1, first, zgl):
        @pl.when(cid == 0)
        def _():
            scan_ref(w1_ref, shift, mask, pshift, pre_b, pre_t, off0, off1,
                     first, zgl)

        @pl.when(cid == 1)
        def _():
            scan_ref(w2_ref, shift, mask, pshift, pre_b, pre_t, off0, off1,
                     first, zgl)

    def fold_publish(nb, offs):
        for rank, off in offs:
            @plsc.parallel_loop(0, nb // 16, 1, unroll=4)
            def _(g, off=off, rank=rank):
                acc = zeros16
                for lane in range(16):
                    acc = acc + hist[pl.ds(lane * _HSTRIDE + off + g * 16, 16)]
                pub[pl.ds(rank * 2048 + g * 16, 16)] = acc
            if nb < 2048:
                @plsc.parallel_loop(0, (2048 - nb) // 16, 1, unroll=4)
                def _(g, rank=rank):
                    pub[pl.ds(rank * 2048 + nb + g * 16, 16)] = zeros16
        pltpu.sync_copy(pub, spub.at[sid])

    def merge_find(states, nbits, shared=False):
        w = 128
        plsc.subcore_barrier()
        for rank, stot in ((0, stot0), (1, stot1)):
            col = pl.multiple_of((0 if shared else rank * 2048) + sid * w, 128)
            pltpu.sync_copy(spub.at[:, pl.ds(col, w)], tmp)
            s_tot = jnp.int32(0)
            for g in range(w // 16):
                m = zeros16
                for row in range(16):
                    m = m + tmp[row, pl.ds(g * 16, 16)]
                cs = plsc.cumsum(m) + s_tot
                mraw[rank, pl.ds(g * 16, 16)] = m
                mcs[rank, pl.ds(g * 16, 16)] = cs
                s_tot = s_tot + jnp.sum(m)
            small[...] = jnp.full((16,), s_tot, jnp.int32)
            pltpu.sync_copy(small, stot.at[sid, pl.ds(0, 16)])
        plsc.subcore_barrier()
        for rank, stot in ((0, stot0), (1, stot1)):
            prefix, resid = states[rank]
            pltpu.sync_copy(stot.at[:, pl.ds(0, 16)], tmp16)
            tvec = zeros16
            for j in range(16):
                tvec = jnp.where(lanes == j, tmp16[j], tvec)
            ct = plsc.cumsum(tvec)
            gexcl = ct - tvec
            my_before = jnp.sum(jnp.where(lanes == sid, gexcl, 0))
            s_me = jnp.sum(jnp.where(lanes == sid, tvec, 0))
            is_owner = (my_before < resid) & (my_before + s_me >= resid)
            found = jnp.int32(0)
            bucket = jnp.int32(0)
            cbef = jnp.int32(0)
            for g in range(w // 16):
                cs = mcs[rank, pl.ds(g * 16, 16)]
                m = mraw[rank, pl.ds(g * 16, 16)]
                hit = (my_before + cs) >= resid
                pc = jnp.max(plsc.all_reduce_population_count(hit))
                lane_s = jnp.max(plsc.all_reduce_ffs(hit))
                csl = jnp.sum(jnp.where(lanes == lane_s, cs, 0))
                ml = jnp.sum(jnp.where(lanes == lane_s, m, 0))
                take = (found == 0) & (pc > 0)
                found = jnp.where(take, jnp.int32(1), found)
                bucket = jnp.where(take, sid * w + g * 16 + lane_s, bucket)
                cbef = jnp.where(take, my_before + csl - ml, cbef)
            new_prefix = jax.lax.shift_left(prefix, nbits) | bucket
            new_resid = resid - cbef

            @pl.when(is_owner)
            def _():
                small[...] = jnp.full((16,), new_prefix, jnp.int32)
                pltpu.sync_copy(small, sresp.at[rank, pl.ds(0, 16)])
                small[...] = jnp.full((16,), new_resid, jnp.int32)
                pltpu.sync_copy(small, sresr.at[rank, pl.ds(0, 16)])
        plsc.subcore_barrier()
        out = []
        for rank in range(2):
            pltpu.sync_copy(sresp.at[rank, pl.ds(0, 16)], rr.at[0])
            pltpu.sync_copy(sresr.at[rank, pl.ds(0, 16)], rr.at[1])
            out.append((jnp.max(rr[0]), jnp.max(rr[1])))
        return out

    r_bot = jnp.int32(_KW)
    r_top = jnp.int32(_NW - _KW + 1)
    zero = jnp.int32(0)

    scan(20, 2047, 0, zero, zero, 0, 0, True, 7)
    fold_publish(2048, ((0, 0),))
    st1 = merge_find(((zero, r_bot), (zero, r_top)), 11, shared=True)

    scan(9, 2047, 20, st1[0][0], st1[1][0], 0, 2048, False, 8)
    fold_publish(2048, ((0, 0), (1, 2048)))
    st2 = merge_find(st1, 11)

    scan(0, 511, 9, st2[0][0], st2[1][0], 0, 512, False, 6)
    fold_publish(512, ((0, 0), (1, 512)))
    st3 = merge_find(st2, 9)

    @pl.when(sid == 0)
    def _():
        ov = jnp.where(lanes == 0, jnp.full((16,), st3[0][0], jnp.int32), zeros16)
        ov = jnp.where(lanes == 1, jnp.full((16,), st3[1][0], jnp.int32), ov)
        small[...] = ov
        pltpu.sync_copy(small, out_ref.at[cid])


def _sc_select(w1_flat, w2_flat):
    mesh = plsc.VectorSubcoreMesh(core_axis_name="c", subcore_axis_name="s")
    return pl.kernel(
        _sc_select_body,
        out_type=jax.ShapeDtypeStruct((2, 16), jnp.int32),
        mesh=mesh,
        compiler_params=pltpu.CompilerParams(
            needs_layout_passes=False, use_tc_tiling_on_sc=False),
        scratch_types=[
            pltpu.VMEM((2, _CHUNK), jnp.float32),
            pltpu.VMEM((65536,), jnp.int32),
            pltpu.VMEM((4096,), jnp.int32),
            pltpu.VMEM((16, 128), jnp.int32),
            pltpu.VMEM((16, 16), jnp.int32),
            pltpu.VMEM((2, 128), jnp.int32),
            pltpu.VMEM((2, 128), jnp.int32),
            pltpu.VMEM((16,), jnp.int32),
            pltpu.VMEM((2, 16), jnp.int32),
            pltpu.VMEM_SHARED((16, 4096), jnp.int32),
            pltpu.VMEM_SHARED((16, 128), jnp.int32),
            pltpu.VMEM_SHARED((16, 128), jnp.int32),
            pltpu.VMEM_SHARED((2, 128), jnp.int32),
            pltpu.VMEM_SHARED((2, 128), jnp.int32),
            pltpu.SemaphoreType.DMA,
            pltpu.SemaphoreType.DMA,
        ],
    )(w1_flat, w2_flat)


def _select2(w_ref, r_bot, r_top):
    def bits():
        return jax.lax.bitcast_convert_type(w_ref[...], jnp.int32) & _ABS_MASK

    def body(_, carry):
        lo_b, hi_b, lo_t, hi_t = carry
        mid_b = (lo_b + hi_b) >> 1
        mid_t = (lo_t + hi_t) >> 1
        b = bits()
        c_b = jnp.sum((b <= mid_b).astype(jnp.int32))
        c_t = jnp.sum((b <= mid_t).astype(jnp.int32))
        ge_b = c_b >= r_bot
        ge_t = c_t >= r_top
        return (
            jnp.where(ge_b, lo_b, mid_b),
            jnp.where(ge_b, mid_b, hi_b),
            jnp.where(ge_t, lo_t, mid_t),
            jnp.where(ge_t, mid_t, hi_t),
        )

    init = (jnp.int32(-1), jnp.int32(0x7FFFFFFF), jnp.int32(-1), jnp.int32(0x7FFFFFFF))
    _, hi_b, _, hi_t = jax.lax.fori_loop(0, 31, body, init)
    return hi_b, hi_t


def _apply_pert(w_ref, hi_b, hi_t, out_dtype):
    b = jax.lax.bitcast_convert_type(w_ref[...], jnp.int32) & _ABS_MASK
    bot = (b <= hi_b).astype(jnp.float32)
    top = (b >= hi_t).astype(jnp.float32)
    return (w_ref[...] * (1.0 + _SCALE * (bot - top))).astype(out_dtype)


def _pert_body(w1_ref, b1_ref, w2_ref, b2_ref, thr_ref,
               pw1_ref, pb1_ref, pw2_ref, pb2_ref):
    for b_ref, p_ref, n in ((b1_ref, pb1_ref, _DF), (b2_ref, pb2_ref, _DM)):
        k = max(1, int(n * _SPARSITY / 2))
        hi_b, hi_t = _select2(b_ref, jnp.int32(k), jnp.int32(n - k + 1))
        p_ref[...] = _apply_pert(b_ref, hi_b, hi_t, jnp.float32)
    for i, (w_ref, p_ref) in enumerate(((w1_ref, pw1_ref), (w2_ref, pw2_ref))):
        p_ref[...] = _apply_pert(w_ref, thr_ref[i, 0], thr_ref[i, 1], jnp.bfloat16)


def _clean_body(
    x_ref, g_ref, bt_ref, wr_ref, br_ref,
    w1_ref, b1_ref, w2_ref, b2_ref,
    oa_ref, p8_ref,
):
    xb = x_ref[...]
    m = jnp.mean(xb, axis=-1, keepdims=True)
    xc = xb - m
    v = jnp.mean(xc * xc, axis=-1, keepdims=True)
    h = xc * jax.lax.rsqrt(v + 1e-5) * g_ref[...] + bt_ref[...]
    logits = (
        jnp.dot(h.astype(jnp.bfloat16), wr_ref[...].astype(jnp.bfloat16),
                preferred_element_type=jnp.float32)
        + br_ref[...]
    )
    mx = jnp.max(logits, axis=-1, keepdims=True)
    e = jnp.exp(logits - mx)
    p0 = e[:, :1] / jnp.sum(e, axis=-1, keepdims=True)
    hc = jax.nn.gelu(
        jnp.dot(xb.astype(jnp.bfloat16), w1_ref[...],
                preferred_element_type=jnp.float32) + b1_ref[...]
    )
    oa_ref[...] = (
        jnp.dot((p0 * hc).astype(jnp.bfloat16), w2_ref[...],
                preferred_element_type=jnp.float32)
        + p0 * b2_ref[...]
    )
    p8_ref[...] = jnp.broadcast_to(p0, p0.shape[:1] + (_NE,))


def _pert_mlp_body(
    x_ref, p8_ref, oa_ref, pw1_ref, pb1_ref, pw2_ref, pb2_ref, o_ref,
):
    pr = 1.0 - p8_ref[:, :1]
    hp = jax.nn.gelu(
        jnp.dot(x_ref[...].astype(jnp.bfloat16), pw1_ref[...],
                preferred_element_type=jnp.float32) + pb1_ref[...]
    )
    o_ref[...] = (
        oa_ref[...]
        + jnp.dot((pr * hp).astype(jnp.bfloat16), pw2_ref[...],
                  preferred_element_type=jnp.float32)
        + pr * pb2_ref[...]
    )


def kernel(x, ln_g, ln_b, Wr, br, W1, b1, W2, b2):
    thr = _sc_select(W1.reshape(-1), W2.reshape(-1))

    b1r = b1.reshape(1, _DF)
    b2r = b2.reshape(1, _DM)
    full = lambda shape: pl.BlockSpec(shape, lambda i: (0, 0))
    tok = lambda cols: pl.BlockSpec((_TB, cols), lambda i: (i, 0))

    out_a, p8 = pl.pallas_call(
        _clean_body,
        grid=(_T // _TB,),
        in_specs=[
            tok(_DM),
            full((1, _DM)),
            full((1, _DM)),
            full((_DM, _NE)),
            full((1, _NE)),
            full((_DM, _DF)),
            full((1, _DF)),
            full((_DF, _DM)),
            full((1, _DM)),
        ],
        out_specs=[tok(_DM), tok(_NE)],
        out_shape=[
            jax.ShapeDtypeStruct((_T, _DM), jnp.float32),
            jax.ShapeDtypeStruct((_T, _NE), jnp.float32),
        ],
    )(
        x, ln_g.reshape(1, _DM), ln_b.reshape(1, _DM), Wr, br.reshape(1, _NE),
        W1.astype(jnp.bfloat16), b1r, W2.astype(jnp.bfloat16), b2r,
    )

    pW1, pb1, pW2, pb2 = pl.pallas_call(
        _pert_body,
        out_shape=[
            jax.ShapeDtypeStruct((_DM, _DF), jnp.bfloat16),
            jax.ShapeDtypeStruct((1, _DF), jnp.float32),
            jax.ShapeDtypeStruct((_DF, _DM), jnp.bfloat16),
            jax.ShapeDtypeStruct((1, _DM), jnp.float32),
        ],
    )(W1, b1r, W2, b2r, thr)

    out = pl.pallas_call(
        _pert_mlp_body,
        grid=(_T // _TB,),
        in_specs=[
            tok(_DM),
            tok(_NE),
            tok(_DM),
            full((_DM, _DF)),
            full((1, _DF)),
            full((_DF, _DM)),
            full((1, _DM)),
        ],
        out_specs=tok(_DM),
        out_shape=jax.ShapeDtypeStruct((_T, _DM), jnp.float32),
    )(x, p8, out_a, pW1, pb1, pW2, pb2)
    return out


# speedup vs baseline: 1.7073x; 1.0304x over previous
import functools

import jax
import jax.numpy as jnp
from jax.experimental import pallas as pl
from jax.experimental.pallas import tpu as pltpu
from jax.experimental.pallas import tpu_sc as plsc

_NE = 8
_DM = 768
_DF = 3072
_SCALE = 0.8
_SPARSITY = 0.1
_T = 4096
_TB = 512

_ABS_MASK = 0x7FFFFFFF
_NW = _DM * _DF
_KW = max(1, int(_NW * _SPARSITY / 2))
_NTILE = 16
_CHUNK = 8192
_PER_TILE = _NW // _NTILE
_NCHUNK = _PER_TILE // _CHUNK
_HSTRIDE = 4096


def _sc_select_body(w1_ref, w2_ref, out_ref,
                    buf, hist, pub, tmp, tmp16, mcs, mraw, small, rr,
                    spub, stot0, stot1, sresp, sresr, sem0, sem1):
    cid = jax.lax.axis_index("c")
    sid = jax.lax.axis_index("s")
    lanes = jax.lax.broadcasted_iota(jnp.int32, (16,), 0)
    ones = jnp.ones((16,), jnp.int32)
    zeros16 = jnp.zeros((16,), jnp.int32)
    sems = (sem0, sem1)

    def scan_ref(w_ref, shift, mask, pshift, pre_b, pre_t, off0, off1, first, zgl):
        @plsc.parallel_loop(0, 16 << zgl, 1, unroll=8)
        def _(g):
            lane = jax.lax.shift_right_logical(g, zgl)
            within = (g & ((1 << zgl) - 1)) * 16
            hist[pl.ds(lane * _HSTRIDE + within, 16)] = zeros16
        base = sid * _PER_TILE

        def chunk_slice(k):
            return w_ref.at[pl.ds(base + k * _CHUNK, _CHUNK)]

        def process(cur):
            @plsc.parallel_loop(0, _CHUNK // 16, 1, unroll=8)
            def _(g):
                v = buf[cur, pl.ds(g * 16, 16)]
                bits = jax.lax.bitcast_convert_type(v, jnp.int32) & _ABS_MASK
                digit = jax.lax.shift_right_logical(bits, shift) & mask
                if first:
                    plsc.addupdate_scatter(hist, [lanes * _HSTRIDE + digit], ones,
                                           mask=jnp.full((16,), True))
                else:
                    pfx = jax.lax.shift_right_logical(bits, pshift)
                    plsc.addupdate_scatter(
                        hist, [lanes * _HSTRIDE + off0 + digit], ones,
                        mask=pfx == pre_b)
                    plsc.addupdate_scatter(
                        hist, [lanes * _HSTRIDE + off1 + digit], ones,
                        mask=pfx == pre_t)

        pltpu.async_copy(chunk_slice(0), buf.at[0], sems[0])
        pltpu.async_copy(chunk_slice(1), buf.at[1], sems[1])

        def chunk_body(j, _):
            pltpu.make_async_copy(chunk_slice(2 * j), buf.at[0], sems[0]).wait()
            process(0)

            @pl.when(j < _NCHUNK // 2 - 1)
            def _():
                pltpu.async_copy(chunk_slice(2 * j + 2), buf.at[0], sems[0])
            pltpu.make_async_copy(chunk_slice(2 * j + 1), buf.at[1], sems[1]).wait()
            process(1)

            @pl.when(j < _NCHUNK // 2 - 1)
            def _():
                pltpu.async_copy(chunk_slice(2 * j + 3), buf.at[1], sems[1])
            return 0
        jax.lax.fori_loop(0, _NCHUNK // 2, chunk_body, 0)

    def scan(shift, mask, pshift, pre_b, pre_t, off0, off1, first, zgl):
        @pl.when(cid == 0)
        def _():
            scan_ref(w1_ref, shift, mask, pshift, pre_b, pre_t, off0, off1,
                     first, zgl)

        @pl.when(cid == 1)
        def _():
            scan_ref(w2_ref, shift, mask, pshift, pre_b, pre_t, off0, off1,
                     first, zgl)

    def fold_publish(nb, offs):
        for rank, off in offs:
            @plsc.parallel_loop(0, nb // 16, 1, unroll=4)
            def _(g, off=off, rank=rank):
                acc = zeros16
                for lane in range(16):
                    acc = acc + hist[pl.ds(lane * _HSTRIDE + off + g * 16, 16)]
                pub[pl.ds(rank * 2048 + g * 16, 16)] = acc
            if nb < 2048:
                @plsc.parallel_loop(0, (2048 - nb) // 16, 1, unroll=4)
                def _(g, rank=rank):
                    pub[pl.ds(rank * 2048 + nb + g * 16, 16)] = zeros16
        pltpu.sync_copy(pub, spub.at[sid])

    def merge_find(states, nbits, shared=False):
        w = 128
        plsc.subcore_barrier()
        for rank, stot in ((0, stot0), (1, stot1)):
            col = pl.multiple_of((0 if shared else rank * 2048) + sid * w, 128)
            pltpu.sync_copy(spub.at[:, pl.ds(col, w)], tmp)
            s_tot = jnp.int32(0)
            for g in range(w // 16):
                m = zeros16
                for row in range(16):
                    m = m + tmp[row, pl.ds(g * 16, 16)]
                cs = plsc.cumsum(m) + s_tot
                mraw[rank, pl.ds(g * 16, 16)] = m
                mcs[rank, pl.ds(g * 16, 16)] = cs
                s_tot = s_tot + jnp.sum(m)
            small[...] = jnp.full((16,), s_tot, jnp.int32)
            pltpu.sync_copy(small, stot.at[sid, pl.ds(0, 16)])
        plsc.subcore_barrier()
        for rank, stot in ((0, stot0), (1, stot1)):
            prefix, resid = states[rank]
            pltpu.sync_copy(stot.at[:, pl.ds(0, 16)], tmp16)
            tvec = zeros16
            for j in range(16):
                tvec = jnp.where(lanes == j, tmp16[j], tvec)
            ct = plsc.cumsum(tvec)
            gexcl = ct - tvec
            my_before = jnp.sum(jnp.where(lanes == sid, gexcl, 0))
            s_me = jnp.sum(jnp.where(lanes == sid, tvec, 0))
            is_owner = (my_before < resid) & (my_before + s_me >= resid)
            found = jnp.int32(0)
            bucket = jnp.int32(0)
            cbef = jnp.int32(0)
            for g in range(w // 16):
                cs = mcs[rank, pl.ds(g * 16, 16)]
                m = mraw[rank, pl.ds(g * 16, 16)]
                hit = (my_before + cs) >= resid
                pc = jnp.max(plsc.all_reduce_population_count(hit))
                lane_s = jnp.max(plsc.all_reduce_ffs(hit))
                csl = jnp.sum(jnp.where(lanes == lane_s, cs, 0))
                ml = jnp.sum(jnp.where(lanes == lane_s, m, 0))
                take = (found == 0) & (pc > 0)
                found = jnp.where(take, jnp.int32(1), found)
                bucket = jnp.where(take, sid * w + g * 16 + lane_s, bucket)
                cbef = jnp.where(take, my_before + csl - ml, cbef)
            new_prefix = jax.lax.shift_left(prefix, nbits) | bucket
            new_resid = resid - cbef

            @pl.when(is_owner)
            def _():
                small[...] = jnp.full((16,), new_prefix, jnp.int32)
                pltpu.sync_copy(small, sresp.at[rank, pl.ds(0, 16)])
                small[...] = jnp.full((16,), new_resid, jnp.int32)
                pltpu.sync_copy(small, sresr.at[rank, pl.ds(0, 16)])
        plsc.subcore_barrier()
        out = []
        for rank in range(2):
            pltpu.sync_copy(sresp.at[rank, pl.ds(0, 16)], rr.at[0])
            pltpu.sync_copy(sresr.at[rank, pl.ds(0, 16)], rr.at[1])
            out.append((jnp.max(rr[0]), jnp.max(rr[1])))
        return out

    r_bot = jnp.int32(_KW)
    r_top = jnp.int32(_NW - _KW + 1)
    zero = jnp.int32(0)

    scan(20, 2047, 0, zero, zero, 0, 0, True, 7)
    fold_publish(2048, ((0, 0),))
    st1 = merge_find(((zero, r_bot), (zero, r_top)), 11, shared=True)

    scan(9, 2047, 20, st1[0][0], st1[1][0], 0, 2048, False, 8)
    fold_publish(2048, ((0, 0), (1, 2048)))
    st2 = merge_find(st1, 11)

    scan(0, 511, 9, st2[0][0], st2[1][0], 0, 512, False, 6)
    fold_publish(512, ((0, 0), (1, 512)))
    st3 = merge_find(st2, 9)

    @pl.when(sid == 0)
    def _():
        ov = jnp.where(lanes == 0, jnp.full((16,), st3[0][0], jnp.int32), zeros16)
        ov = jnp.where(lanes == 1, jnp.full((16,), st3[1][0], jnp.int32), ov)
        small[...] = ov
        pltpu.sync_copy(small, out_ref.at[cid])


def _sc_select(w1_flat, w2_flat):
    mesh = plsc.VectorSubcoreMesh(core_axis_name="c", subcore_axis_name="s")
    return pl.kernel(
        _sc_select_body,
        out_type=jax.ShapeDtypeStruct((2, 16), jnp.int32),
        mesh=mesh,
        compiler_params=pltpu.CompilerParams(
            needs_layout_passes=False, use_tc_tiling_on_sc=False),
        scratch_types=[
            pltpu.VMEM((2, _CHUNK), jnp.float32),
            pltpu.VMEM((65536,), jnp.int32),
            pltpu.VMEM((4096,), jnp.int32),
            pltpu.VMEM((16, 128), jnp.int32),
            pltpu.VMEM((16, 16), jnp.int32),
            pltpu.VMEM((2, 128), jnp.int32),
            pltpu.VMEM((2, 128), jnp.int32),
            pltpu.VMEM((16,), jnp.int32),
            pltpu.VMEM((2, 16), jnp.int32),
            pltpu.VMEM_SHARED((16, 4096), jnp.int32),
            pltpu.VMEM_SHARED((16, 128), jnp.int32),
            pltpu.VMEM_SHARED((16, 128), jnp.int32),
            pltpu.VMEM_SHARED((2, 128), jnp.int32),
            pltpu.VMEM_SHARED((2, 128), jnp.int32),
            pltpu.SemaphoreType.DMA,
            pltpu.SemaphoreType.DMA,
        ],
    )(w1_flat, w2_flat)


def _select2(w_ref, r_bot, r_top):
    def bits():
        return jax.lax.bitcast_convert_type(w_ref[...], jnp.int32) & _ABS_MASK

    def body(_, carry):
        lo_b, hi_b, lo_t, hi_t = carry
        mid_b = (lo_b + hi_b) >> 1
        mid_t = (lo_t + hi_t) >> 1
        b = bits()
        c_b = jnp.sum((b <= mid_b).astype(jnp.int32))
        c_t = jnp.sum((b <= mid_t).astype(jnp.int32))
        ge_b = c_b >= r_bot
        ge_t = c_t >= r_top
        return (
            jnp.where(ge_b, lo_b, mid_b),
            jnp.where(ge_b, mid_b, hi_b),
            jnp.where(ge_t, lo_t, mid_t),
            jnp.where(ge_t, mid_t, hi_t),
        )

    init = (jnp.int32(-1), jnp.int32(0x7FFFFFFF), jnp.int32(-1), jnp.int32(0x7FFFFFFF))
    _, hi_b, _, hi_t = jax.lax.fori_loop(0, 31, body, init)
    return hi_b, hi_t


def _apply_pert(w_ref, hi_b, hi_t, out_dtype):
    b = jax.lax.bitcast_convert_type(w_ref[...], jnp.int32) & _ABS_MASK
    bot = (b <= hi_b).astype(jnp.float32)
    top = (b >= hi_t).astype(jnp.float32)
    return (w_ref[...] * (1.0 + _SCALE * (bot - top))).astype(out_dtype)


def _clean_body(
    x_ref, g_ref, bt_ref, wr_ref, br_ref,
    w1_ref, b1_ref, w2_ref, b2_ref,
    oa_ref, p8_ref,
):
    xb = x_ref[...]
    m = jnp.mean(xb, axis=-1, keepdims=True)
    xc = xb - m
    v = jnp.mean(xc * xc, axis=-1, keepdims=True)
    h = xc * jax.lax.rsqrt(v + 1e-5) * g_ref[...] + bt_ref[...]
    logits = (
        jnp.dot(h.astype(jnp.bfloat16), wr_ref[...].astype(jnp.bfloat16),
                preferred_element_type=jnp.float32)
        + br_ref[...]
    )
    mx = jnp.max(logits, axis=-1, keepdims=True)
    e = jnp.exp(logits - mx)
    p0 = e[:, :1] / jnp.sum(e, axis=-1, keepdims=True)
    hc = jax.nn.gelu(
        jnp.dot(xb.astype(jnp.bfloat16), w1_ref[...],
                preferred_element_type=jnp.float32) + b1_ref[...]
    )
    oa_ref[...] = (
        jnp.dot((p0 * hc).astype(jnp.bfloat16), w2_ref[...],
                preferred_element_type=jnp.float32)
        + p0 * b2_ref[...]
    )
    p8_ref[...] = jnp.broadcast_to(p0, p0.shape[:1] + (_NE,))


def _pert_mlp_body(
    x_ref, p8_ref, oa_ref, w1_ref, b1_ref, w2_ref, b2_ref, thr_ref, o_ref,
    pw1_s, pb1_s, pw2_s, pb2_s,
):
    @pl.when(pl.program_id(0) == 0)
    def _():
        for b_ref, p_ref, n in ((b1_ref, pb1_s, _DF), (b2_ref, pb2_s, _DM)):
            k = max(1, int(n * _SPARSITY / 2))
            hi_b, hi_t = _select2(b_ref, jnp.int32(k), jnp.int32(n - k + 1))
            p_ref[...] = _apply_pert(b_ref, hi_b, hi_t, jnp.float32)
        pw1_s[...] = _apply_pert(w1_ref, thr_ref[0, 0], thr_ref[0, 1], jnp.bfloat16)
        pw2_s[...] = _apply_pert(w2_ref, thr_ref[1, 0], thr_ref[1, 1], jnp.bfloat16)

    pr = 1.0 - p8_ref[:, :1]
    hp = jax.nn.gelu(
        jnp.dot(x_ref[...].astype(jnp.bfloat16), pw1_s[...],
                preferred_element_type=jnp.float32) + pb1_s[...]
    )
    o_ref[...] = (
        oa_ref[...]
        + jnp.dot((pr * hp).astype(jnp.bfloat16), pw2_s[...],
                  preferred_element_type=jnp.float32)
        + pr * pb2_s[...]
    )


def kernel(x, ln_g, ln_b, Wr, br, W1, b1, W2, b2):
    thr = _sc_select(W1.reshape(-1), W2.reshape(-1))

    b1r = b1.reshape(1, _DF)
    b2r = b2.reshape(1, _DM)
    full = lambda shape: pl.BlockSpec(shape, lambda i: (0, 0))
    tok = lambda cols: pl.BlockSpec((_TB, cols), lambda i: (i, 0))

    out_a, p8 = pl.pallas_call(
        _clean_body,
        grid=(_T // _TB,),
        in_specs=[
            tok(_DM),
            full((1, _DM)),
            full((1, _DM)),
            full((_DM, _NE)),
            full((1, _NE)),
            full((_DM, _DF)),
            full((1, _DF)),
            full((_DF, _DM)),
            full((1, _DM)),
        ],
        out_specs=[tok(_DM), tok(_NE)],
        out_shape=[
            jax.ShapeDtypeStruct((_T, _DM), jnp.float32),
            jax.ShapeDtypeStruct((_T, _NE), jnp.float32),
        ],
    )(
        x, ln_g.reshape(1, _DM), ln_b.reshape(1, _DM), Wr, br.reshape(1, _NE),
        W1.astype(jnp.bfloat16), b1r, W2.astype(jnp.bfloat16), b2r,
    )

    out = pl.pallas_call(
        _pert_mlp_body,
        grid=(_T // _TB,),
        in_specs=[
            tok(_DM),
            tok(_NE),
            tok(_DM),
            full((_DM, _DF)),
            full((1, _DF)),
            full((_DF, _DM)),
            full((1, _DM)),
            full((2, 16)),
        ],
        out_specs=tok(_DM),
        out_shape=jax.ShapeDtypeStruct((_T, _DM), jnp.float32),
        scratch_shapes=[
            pltpu.VMEM((_DM, _DF), jnp.bfloat16),
            pltpu.VMEM((1, _DF), jnp.float32),
            pltpu.VMEM((_DF, _DM), jnp.bfloat16),
            pltpu.VMEM((1, _DM), jnp.float32),
        ],
    )(x, p8, out_a, W1, b1r, W2, b2r, thr)
    return out


# speedup vs baseline: 1.7190x; 1.0069x over previous
import functools

import jax
import jax.numpy as jnp
from jax.experimental import pallas as pl
from jax.experimental.pallas import tpu as pltpu
from jax.experimental.pallas import tpu_sc as plsc

_NE = 8
_DM = 768
_DF = 3072
_SCALE = 0.8
_SPARSITY = 0.1
_T = 4096
_TB = 512

_ABS_MASK = 0x7FFFFFFF
_NW = _DM * _DF
_KW = max(1, int(_NW * _SPARSITY / 2))
_NTILE = 16
_CHUNK = 12288
_PER_TILE = _NW // _NTILE
_NCHUNK = _PER_TILE // _CHUNK
_HSTRIDE = 4096


def _sc_select_body(w1_ref, w2_ref, out_ref,
                    buf, hist, pub, tmp, tmp16, mcs, mraw, small, rr,
                    spub, stot0, stot1, sresp, sresr, sem0, sem1):
    cid = jax.lax.axis_index("c")
    sid = jax.lax.axis_index("s")
    lanes = jax.lax.broadcasted_iota(jnp.int32, (16,), 0)
    ones = jnp.ones((16,), jnp.int32)
    zeros16 = jnp.zeros((16,), jnp.int32)
    sems = (sem0, sem1)

    def scan_ref(w_ref, shift, mask, pshift, pre_b, pre_t, off0, off1, first, zgl):
        @plsc.parallel_loop(0, 16 << zgl, 1, unroll=8)
        def _(g):
            lane = jax.lax.shift_right_logical(g, zgl)
            within = (g & ((1 << zgl) - 1)) * 16
            hist[pl.ds(lane * _HSTRIDE + within, 16)] = zeros16
        base = sid * _PER_TILE

        def chunk_slice(k):
            return w_ref.at[pl.ds(base + k * _CHUNK, _CHUNK)]

        def process(cur):
            @plsc.parallel_loop(0, _CHUNK // 16, 1, unroll=8)
            def _(g):
                v = buf[cur, pl.ds(g * 16, 16)]
                bits = jax.lax.bitcast_convert_type(v, jnp.int32) & _ABS_MASK
                digit = jax.lax.shift_right_logical(bits, shift) & mask
                if first:
                    plsc.addupdate_scatter(hist, [lanes * _HSTRIDE + digit], ones,
                                           mask=jnp.full((16,), True))
                else:
                    pfx = jax.lax.shift_right_logical(bits, pshift)
                    plsc.addupdate_scatter(
                        hist, [lanes * _HSTRIDE + off0 + digit], ones,
                        mask=pfx == pre_b)
                    plsc.addupdate_scatter(
                        hist, [lanes * _HSTRIDE + off1 + digit], ones,
                        mask=pfx == pre_t)

        pltpu.async_copy(chunk_slice(0), buf.at[0], sems[0])
        pltpu.async_copy(chunk_slice(1), buf.at[1], sems[1])

        def chunk_body(j, _):
            pltpu.make_async_copy(chunk_slice(2 * j), buf.at[0], sems[0]).wait()
            process(0)

            @pl.when(j < _NCHUNK // 2 - 1)
            def _():
                pltpu.async_copy(chunk_slice(2 * j + 2), buf.at[0], sems[0])
            pltpu.make_async_copy(chunk_slice(2 * j + 1), buf.at[1], sems[1]).wait()
            process(1)

            @pl.when(j < _NCHUNK // 2 - 1)
            def _():
                pltpu.async_copy(chunk_slice(2 * j + 3), buf.at[1], sems[1])
            return 0
        jax.lax.fori_loop(0, _NCHUNK // 2, chunk_body, 0)

    def scan(shift, mask, pshift, pre_b, pre_t, off0, off1, first, zgl):
        @pl.when(cid == 0)
        def _():
            scan_ref(w1_ref, shift, mask, pshift, pre_b, pre_t, off0, off1,
                     first, zgl)

        @pl.when(cid == 1)
        def _():
            scan_ref(w2_ref, shift, mask, pshift, pre_b, pre_t, off0, off1,
                     first, zgl)

    def fold_publish(nb, offs):
        for rank, off in offs:
            @plsc.parallel_loop(0, nb // 16, 1, unroll=4)
            def _(g, off=off, rank=rank):
                acc = zeros16
                for lane in range(16):
                    acc = acc + hist[pl.ds(lane * _HSTRIDE + off + g * 16, 16)]
                pub[pl.ds(rank * 2048 + g * 16, 16)] = acc
            if nb < 2048:
                @plsc.parallel_loop(0, (2048 - nb) // 16, 1, unroll=4)
                def _(g, rank=rank):
                    pub[pl.ds(rank * 2048 + nb + g * 16, 16)] = zeros16
        pltpu.sync_copy(pub, spub.at[sid])

    def merge_find(states, nbits, shared=False):
        w = 128
        plsc.subcore_barrier()
        for rank, stot in ((0, stot0), (1, stot1)):
            col = pl.multiple_of((0 if shared else rank * 2048) + sid * w, 128)
            pltpu.sync_copy(spub.at[:, pl.ds(col, w)], tmp)
            s_tot = jnp.int32(0)
            for g in range(w // 16):
                m = zeros16
                for row in range(16):
                    m = m + tmp[row, pl.ds(g * 16, 16)]
                cs = plsc.cumsum(m) + s_tot
                mraw[rank, pl.ds(g * 16, 16)] = m
                mcs[rank, pl.ds(g * 16, 16)] = cs
                s_tot = s_tot + jnp.sum(m)
            small[...] = jnp.full((16,), s_tot, jnp.int32)
            pltpu.sync_copy(small, stot.at[sid, pl.ds(0, 16)])
        plsc.subcore_barrier()
        for rank, stot in ((0, stot0), (1, stot1)):
            prefix, resid = states[rank]
            pltpu.sync_copy(stot.at[:, pl.ds(0, 16)], tmp16)
            tvec = zeros16
            for j in range(16):
                tvec = jnp.where(lanes == j, tmp16[j], tvec)
            ct = plsc.cumsum(tvec)
            gexcl = ct - tvec
            my_before = jnp.sum(jnp.where(lanes == sid, gexcl, 0))
            s_me = jnp.sum(jnp.where(lanes == sid, tvec, 0))
            is_owner = (my_before < resid) & (my_before + s_me >= resid)
            found = jnp.int32(0)
            bucket = jnp.int32(0)
            cbef = jnp.int32(0)
            for g in range(w // 16):
                cs = mcs[rank, pl.ds(g * 16, 16)]
                m = mraw[rank, pl.ds(g * 16, 16)]
                hit = (my_before + cs) >= resid
                pc = jnp.max(plsc.all_reduce_population_count(hit))
                lane_s = jnp.max(plsc.all_reduce_ffs(hit))
                csl = jnp.sum(jnp.where(lanes == lane_s, cs, 0))
                ml = jnp.sum(jnp.where(lanes == lane_s, m, 0))
                take = (found == 0) & (pc > 0)
                found = jnp.where(take, jnp.int32(1), found)
                bucket = jnp.where(take, sid * w + g * 16 + lane_s, bucket)
                cbef = jnp.where(take, my_before + csl - ml, cbef)
            new_prefix = jax.lax.shift_left(prefix, nbits) | bucket
            new_resid = resid - cbef

            @pl.when(is_owner)
            def _():
                small[...] = jnp.full((16,), new_prefix, jnp.int32)
                pltpu.sync_copy(small, sresp.at[rank, pl.ds(0, 16)])
                small[...] = jnp.full((16,), new_resid, jnp.int32)
                pltpu.sync_copy(small, sresr.at[rank, pl.ds(0, 16)])
        plsc.subcore_barrier()
        out = []
        for rank in range(2):
            pltpu.sync_copy(sresp.at[rank, pl.ds(0, 16)], rr.at[0])
            pltpu.sync_copy(sresr.at[rank, pl.ds(0, 16)], rr.at[1])
            out.append((jnp.max(rr[0]), jnp.max(rr[1])))
        return out

    r_bot = jnp.int32(_KW)
    r_top = jnp.int32(_NW - _KW + 1)
    zero = jnp.int32(0)

    scan(20, 2047, 0, zero, zero, 0, 0, True, 7)
    fold_publish(2048, ((0, 0),))
    st1 = merge_find(((zero, r_bot), (zero, r_top)), 11, shared=True)

    scan(9, 2047, 20, st1[0][0], st1[1][0], 0, 2048, False, 8)
    fold_publish(2048, ((0, 0), (1, 2048)))
    st2 = merge_find(st1, 11)

    scan(0, 511, 9, st2[0][0], st2[1][0], 0, 512, False, 6)
    fold_publish(512, ((0, 0), (1, 512)))
    st3 = merge_find(st2, 9)

    @pl.when(sid == 0)
    def _():
        ov = jnp.where(lanes == 0, jnp.full((16,), st3[0][0], jnp.int32), zeros16)
        ov = jnp.where(lanes == 1, jnp.full((16,), st3[1][0], jnp.int32), ov)
        small[...] = ov
        pltpu.sync_copy(small, out_ref.at[cid])


def _sc_select(w1_flat, w2_flat):
    mesh = plsc.VectorSubcoreMesh(core_axis_name="c", subcore_axis_name="s")
    return pl.kernel(
        _sc_select_body,
        out_type=jax.ShapeDtypeStruct((2, 16), jnp.int32),
        mesh=mesh,
        compiler_params=pltpu.CompilerParams(
            needs_layout_passes=False, use_tc_tiling_on_sc=False),
        scratch_types=[
            pltpu.VMEM((2, _CHUNK), jnp.float32),
            pltpu.VMEM((65536,), jnp.int32),
            pltpu.VMEM((4096,), jnp.int32),
            pltpu.VMEM((16, 128), jnp.int32),
            pltpu.VMEM((16, 16), jnp.int32),
            pltpu.VMEM((2, 128), jnp.int32),
            pltpu.VMEM((2, 128), jnp.int32),
            pltpu.VMEM((16,), jnp.int32),
            pltpu.VMEM((2, 16), jnp.int32),
            pltpu.VMEM_SHARED((16, 4096), jnp.int32),
            pltpu.VMEM_SHARED((16, 128), jnp.int32),
            pltpu.VMEM_SHARED((16, 128), jnp.int32),
            pltpu.VMEM_SHARED((2, 128), jnp.int32),
            pltpu.VMEM_SHARED((2, 128), jnp.int32),
            pltpu.SemaphoreType.DMA,
            pltpu.SemaphoreType.DMA,
        ],
    )(w1_flat, w2_flat)


def _select2(w_ref, r_bot, r_top):
    def bits():
        return jax.lax.bitcast_convert_type(w_ref[...], jnp.int32) & _ABS_MASK

    def body(_, carry):
        lo_b, hi_b, lo_t, hi_t = carry
        mid_b = (lo_b + hi_b) >> 1
        mid_t = (lo_t + hi_t) >> 1
        b = bits()
        c_b = jnp.sum((b <= mid_b).astype(jnp.int32))
        c_t = jnp.sum((b <= mid_t).astype(jnp.int32))
        ge_b = c_b >= r_bot
        ge_t = c_t >= r_top
        return (
            jnp.where(ge_b, lo_b, mid_b),
            jnp.where(ge_b, mid_b, hi_b),
            jnp.where(ge_t, lo_t, mid_t),
            jnp.where(ge_t, mid_t, hi_t),
        )

    init = (jnp.int32(-1), jnp.int32(0x7FFFFFFF), jnp.int32(-1), jnp.int32(0x7FFFFFFF))
    _, hi_b, _, hi_t = jax.lax.fori_loop(0, 31, body, init)
    return hi_b, hi_t


def _apply_pert(w_ref, hi_b, hi_t, out_dtype):
    b = jax.lax.bitcast_convert_type(w_ref[...], jnp.int32) & _ABS_MASK
    bot = (b <= hi_b).astype(jnp.float32)
    top = (b >= hi_t).astype(jnp.float32)
    return (w_ref[...] * (1.0 + _SCALE * (bot - top))).astype(out_dtype)


def _clean_body(
    x_ref, g_ref, bt_ref, wr_ref, br_ref,
    w1_ref, b1_ref, w2_ref, b2_ref,
    oa_ref, p8_ref,
):
    xb = x_ref[...]
    m = jnp.mean(xb, axis=-1, keepdims=True)
    xc = xb - m
    v = jnp.mean(xc * xc, axis=-1, keepdims=True)
    h = xc * jax.lax.rsqrt(v + 1e-5) * g_ref[...] + bt_ref[...]
    logits = (
        jnp.dot(h.astype(jnp.bfloat16), wr_ref[...].astype(jnp.bfloat16),
                preferred_element_type=jnp.float32)
        + br_ref[...]
    )
    mx = jnp.max(logits, axis=-1, keepdims=True)
    e = jnp.exp(logits - mx)
    p0 = e[:, :1] / jnp.sum(e, axis=-1, keepdims=True)
    hc = jax.nn.gelu(
        jnp.dot(xb.astype(jnp.bfloat16), w1_ref[...],
                preferred_element_type=jnp.float32) + b1_ref[...]
    )
    oa_ref[...] = (
        jnp.dot((p0 * hc).astype(jnp.bfloat16), w2_ref[...],
                preferred_element_type=jnp.float32)
        + p0 * b2_ref[...]
    )
    p8_ref[...] = jnp.broadcast_to(p0, p0.shape[:1] + (_NE,))


def _pert_mlp_body(
    x_ref, p8_ref, oa_ref, w1_ref, b1_ref, w2_ref, b2_ref, thr_ref, o_ref,
    pw1_s, pb1_s, pw2_s, pb2_s,
):
    @pl.when(pl.program_id(0) == 0)
    def _():
        for b_ref, p_ref, n in ((b1_ref, pb1_s, _DF), (b2_ref, pb2_s, _DM)):
            k = max(1, int(n * _SPARSITY / 2))
            hi_b, hi_t = _select2(b_ref, jnp.int32(k), jnp.int32(n - k + 1))
            p_ref[...] = _apply_pert(b_ref, hi_b, hi_t, jnp.float32)
        pw1_s[...] = _apply_pert(w1_ref, thr_ref[0, 0], thr_ref[0, 1], jnp.bfloat16)
        pw2_s[...] = _apply_pert(w2_ref, thr_ref[1, 0], thr_ref[1, 1], jnp.bfloat16)

    pr = 1.0 - p8_ref[:, :1]
    hp = jax.nn.gelu(
        jnp.dot(x_ref[...].astype(jnp.bfloat16), pw1_s[...],
                preferred_element_type=jnp.float32) + pb1_s[...]
    )
    o_ref[...] = (
        oa_ref[...]
        + jnp.dot((pr * hp).astype(jnp.bfloat16), pw2_s[...],
                  preferred_element_type=jnp.float32)
        + pr * pb2_s[...]
    )


def kernel(x, ln_g, ln_b, Wr, br, W1, b1, W2, b2):
    thr = _sc_select(W1.reshape(-1), W2.reshape(-1))

    b1r = b1.reshape(1, _DF)
    b2r = b2.reshape(1, _DM)
    full = lambda shape: pl.BlockSpec(shape, lambda i: (0, 0))
    tok = lambda cols: pl.BlockSpec((_TB, cols), lambda i: (i, 0))

    out_a, p8 = pl.pallas_call(
        _clean_body,
        grid=(_T // _TB,),
        in_specs=[
            tok(_DM),
            full((1, _DM)),
            full((1, _DM)),
            full((_DM, _NE)),
            full((1, _NE)),
            full((_DM, _DF)),
            full((1, _DF)),
            full((_DF, _DM)),
            full((1, _DM)),
        ],
        out_specs=[tok(_DM), tok(_NE)],
        out_shape=[
            jax.ShapeDtypeStruct((_T, _DM), jnp.float32),
            jax.ShapeDtypeStruct((_T, _NE), jnp.float32),
        ],
    )(
        x, ln_g.reshape(1, _DM), ln_b.reshape(1, _DM), Wr, br.reshape(1, _NE),
        W1.astype(jnp.bfloat16), b1r, W2.astype(jnp.bfloat16), b2r,
    )

    out = pl.pallas_call(
        _pert_mlp_body,
        grid=(_T // _TB,),
        in_specs=[
            tok(_DM),
            tok(_NE),
            tok(_DM),
            full((_DM, _DF)),
            full((1, _DF)),
            full((_DF, _DM)),
            full((1, _DM)),
            full((2, 16)),
        ],
        out_specs=tok(_DM),
        out_shape=jax.ShapeDtypeStruct((_T, _DM), jnp.float32),
        scratch_shapes=[
            pltpu.VMEM((_DM, _DF), jnp.bfloat16),
            pltpu.VMEM((1, _DF), jnp.float32),
            pltpu.VMEM((_DF, _DM), jnp.bfloat16),
            pltpu.VMEM((1, _DM), jnp.float32),
        ],
    )(x, p8, out_a, W1, b1r, W2, b2r, thr)
    return out


# speedup vs baseline: 1.7216x; 1.0015x over previous
import jax
import jax.numpy as jnp
from jax.experimental import pallas as pl
from jax.experimental.pallas import tpu as pltpu
from jax.experimental.pallas import tpu_sc as plsc

_NE = 8
_DM = 768
_DF = 3072
_SCALE = 0.8
_SPARSITY = 0.1
_T = 4096
_TB = 512

_ABS_MASK = 0x7FFFFFFF
_NW = _DM * _DF
_KW = max(1, int(_NW * _SPARSITY / 2))
_NTILE = 16
_CHUNK = 12288
_PER_TILE = _NW // _NTILE
_NCHUNK = _PER_TILE // _CHUNK
_HSTRIDE = 4096


def _sc_select_body(w1_ref, w2_ref, out_ref,
                    buf, hist, pub, tmp, tmp16, mcs, mraw, small, rr,
                    spub, stot0, stot1, sresp, sresr, sem0, sem1):
    cid = jax.lax.axis_index("c")
    sid = jax.lax.axis_index("s")
    lanes = jax.lax.broadcasted_iota(jnp.int32, (16,), 0)
    ones = jnp.ones((16,), jnp.int32)
    zeros16 = jnp.zeros((16,), jnp.int32)
    sems = (sem0, sem1)

    def scan_ref(w_ref, shift, mask, pshift, pre_b, pre_t, off0, off1, first, zgl):
        @plsc.parallel_loop(0, 16 << zgl, 1, unroll=8)
        def _(g):
            lane = jax.lax.shift_right_logical(g, zgl)
            within = (g & ((1 << zgl) - 1)) * 16
            hist[pl.ds(lane * _HSTRIDE + within, 16)] = zeros16
        base = sid * _PER_TILE

        def chunk_slice(k):
            return w_ref.at[pl.ds(base + k * _CHUNK, _CHUNK)]

        def process(cur):
            @plsc.parallel_loop(0, _CHUNK // 16, 1, unroll=8)
            def _(g):
                v = buf[cur, pl.ds(g * 16, 16)]
                bits = jax.lax.bitcast_convert_type(v, jnp.int32) & _ABS_MASK
                digit = jax.lax.shift_right_logical(bits, shift) & mask
                if first:
                    plsc.addupdate_scatter(hist, [lanes * _HSTRIDE + digit], ones,
                                           mask=jnp.full((16,), True))
                else:
                    pfx = jax.lax.shift_right_logical(bits, pshift)
                    plsc.addupdate_scatter(
                        hist, [lanes * _HSTRIDE + off0 + digit], ones,
                        mask=pfx == pre_b)
                    plsc.addupdate_scatter(
                        hist, [lanes * _HSTRIDE + off1 + digit], ones,
                        mask=pfx == pre_t)

        pltpu.async_copy(chunk_slice(0), buf.at[0], sems[0])
        pltpu.async_copy(chunk_slice(1), buf.at[1], sems[1])

        def chunk_body(j, _):
            pltpu.make_async_copy(chunk_slice(2 * j), buf.at[0], sems[0]).wait()
            process(0)

            @pl.when(j < _NCHUNK // 2 - 1)
            def _():
                pltpu.async_copy(chunk_slice(2 * j + 2), buf.at[0], sems[0])
            pltpu.make_async_copy(chunk_slice(2 * j + 1), buf.at[1], sems[1]).wait()
            process(1)

            @pl.when(j < _NCHUNK // 2 - 1)
            def _():
                pltpu.async_copy(chunk_slice(2 * j + 3), buf.at[1], sems[1])
            return 0
        jax.lax.fori_loop(0, _NCHUNK // 2, chunk_body, 0)

    def scan(shift, mask, pshift, pre_b, pre_t, off0, off1, first, zgl):
        @pl.when(cid == 0)
        def _():
            scan_ref(w1_ref, shift, mask, pshift, pre_b, pre_t, off0, off1,
                     first, zgl)

        @pl.when(cid == 1)
        def _():
            scan_ref(w2_ref, shift, mask, pshift, pre_b, pre_t, off0, off1,
                     first, zgl)

    def fold_publish(nb, offs):
        for rank, off in offs:
            @plsc.parallel_loop(0, nb // 16, 1, unroll=4)
            def _(g, off=off, rank=rank):
                acc = zeros16
                for lane in range(16):
                    acc = acc + hist[pl.ds(lane * _HSTRIDE + off + g * 16, 16)]
                pub[pl.ds(rank * 2048 + g * 16, 16)] = acc
            if nb < 2048:
                @plsc.parallel_loop(0, (2048 - nb) // 16, 1, unroll=4)
                def _(g, rank=rank):
                    pub[pl.ds(rank * 2048 + nb + g * 16, 16)] = zeros16
        pltpu.sync_copy(pub, spub.at[sid])

    def merge_find(states, nbits, shared=False):
        w = 128
        plsc.subcore_barrier()
        for rank, stot in ((0, stot0), (1, stot1)):
            col = pl.multiple_of((0 if shared else rank * 2048) + sid * w, 128)
            pltpu.sync_copy(spub.at[:, pl.ds(col, w)], tmp)
            s_tot = jnp.int32(0)
            for g in range(w // 16):
                m = zeros16
                for row in range(16):
                    m = m + tmp[row, pl.ds(g * 16, 16)]
                cs = plsc.cumsum(m) + s_tot
                mraw[rank, pl.ds(g * 16, 16)] = m
                mcs[rank, pl.ds(g * 16, 16)] = cs
                s_tot = s_tot + jnp.sum(m)
            small[...] = jnp.full((16,), s_tot, jnp.int32)
            pltpu.sync_copy(small, stot.at[sid, pl.ds(0, 16)])
        plsc.subcore_barrier()
        for rank, stot in ((0, stot0), (1, stot1)):
            prefix, resid = states[rank]
            pltpu.sync_copy(stot.at[:, pl.ds(0, 16)], tmp16)
            tvec = zeros16
            for j in range(16):
                tvec = jnp.where(lanes == j, tmp16[j], tvec)
            ct = plsc.cumsum(tvec)
            gexcl = ct - tvec
            my_before = jnp.sum(jnp.where(lanes == sid, gexcl, 0))
            s_me = jnp.sum(jnp.where(lanes == sid, tvec, 0))
            is_owner = (my_before < resid) & (my_before + s_me >= resid)
            found = jnp.int32(0)
            bucket = jnp.int32(0)
            cbef = jnp.int32(0)
            for g in range(w // 16):
                cs = mcs[rank, pl.ds(g * 16, 16)]
                m = mraw[rank, pl.ds(g * 16, 16)]
                hit = (my_before + cs) >= resid
                pc = jnp.max(plsc.all_reduce_population_count(hit))
                lane_s = jnp.max(plsc.all_reduce_ffs(hit))
                csl = jnp.sum(jnp.where(lanes == lane_s, cs, 0))
                ml = jnp.sum(jnp.where(lanes == lane_s, m, 0))
                take = (found == 0) & (pc > 0)
                found = jnp.where(take, jnp.int32(1), found)
                bucket = jnp.where(take, sid * w + g * 16 + lane_s, bucket)
                cbef = jnp.where(take, my_before + csl - ml, cbef)
            new_prefix = jax.lax.shift_left(prefix, nbits) | bucket
            new_resid = resid - cbef

            @pl.when(is_owner)
            def _():
                small[...] = jnp.full((16,), new_prefix, jnp.int32)
                pltpu.sync_copy(small, sresp.at[rank, pl.ds(0, 16)])
                small[...] = jnp.full((16,), new_resid, jnp.int32)
                pltpu.sync_copy(small, sresr.at[rank, pl.ds(0, 16)])
        plsc.subcore_barrier()
        out = []
        for rank in range(2):
            pltpu.sync_copy(sresp.at[rank, pl.ds(0, 16)], rr.at[0])
            pltpu.sync_copy(sresr.at[rank, pl.ds(0, 16)], rr.at[1])
            out.append((jnp.max(rr[0]), jnp.max(rr[1])))
        return out

    r_bot = jnp.int32(_KW)
    r_top = jnp.int32(_NW - _KW + 1)
    zero = jnp.int32(0)

    scan(20, 2047, 0, zero, zero, 0, 0, True, 7)
    fold_publish(2048, ((0, 0),))
    st1 = merge_find(((zero, r_bot), (zero, r_top)), 11, shared=True)

    scan(9, 2047, 20, st1[0][0], st1[1][0], 0, 2048, False, 8)
    fold_publish(2048, ((0, 0), (1, 2048)))
    st2 = merge_find(st1, 11)

    scan(0, 511, 9, st2[0][0], st2[1][0], 0, 512, False, 6)
    fold_publish(512, ((0, 0), (1, 512)))
    st3 = merge_find(st2, 9)

    @pl.when(sid == 0)
    def _():
        ov = jnp.where(lanes == 0, jnp.full((16,), st3[0][0], jnp.int32), zeros16)
        ov = jnp.where(lanes == 1, jnp.full((16,), st3[1][0], jnp.int32), ov)
        small[...] = ov
        pltpu.sync_copy(small, out_ref.at[cid])


def _sc_select(w1_flat, w2_flat):
    mesh = plsc.VectorSubcoreMesh(core_axis_name="c", subcore_axis_name="s")
    return pl.kernel(
        _sc_select_body,
        out_type=jax.ShapeDtypeStruct((2, 16), jnp.int32),
        mesh=mesh,
        compiler_params=pltpu.CompilerParams(
            needs_layout_passes=False, use_tc_tiling_on_sc=False),
        scratch_types=[
            pltpu.VMEM((2, _CHUNK), jnp.float32),
            pltpu.VMEM((65536,), jnp.int32),
            pltpu.VMEM((4096,), jnp.int32),
            pltpu.VMEM((16, 128), jnp.int32),
            pltpu.VMEM((16, 16), jnp.int32),
            pltpu.VMEM((2, 128), jnp.int32),
            pltpu.VMEM((2, 128), jnp.int32),
            pltpu.VMEM((16,), jnp.int32),
            pltpu.VMEM((2, 16), jnp.int32),
            pltpu.VMEM_SHARED((16, 4096), jnp.int32),
            pltpu.VMEM_SHARED((16, 128), jnp.int32),
            pltpu.VMEM_SHARED((16, 128), jnp.int32),
            pltpu.VMEM_SHARED((2, 128), jnp.int32),
            pltpu.VMEM_SHARED((2, 128), jnp.int32),
            pltpu.SemaphoreType.DMA,
            pltpu.SemaphoreType.DMA,
        ],
    )(w1_flat, w2_flat)


def _select2(w_ref, r_bot, r_top):
    def bits():
        return jax.lax.bitcast_convert_type(w_ref[...], jnp.int32) & _ABS_MASK

    def body(_, carry):
        lo_b, hi_b, lo_t, hi_t = carry
        mid_b = (lo_b + hi_b) >> 1
        mid_t = (lo_t + hi_t) >> 1
        b = bits()
        c_b = jnp.sum((b <= mid_b).astype(jnp.int32))
        c_t = jnp.sum((b <= mid_t).astype(jnp.int32))
        ge_b = c_b >= r_bot
        ge_t = c_t >= r_top
        return (
            jnp.where(ge_b, lo_b, mid_b),
            jnp.where(ge_b, mid_b, hi_b),
            jnp.where(ge_t, lo_t, mid_t),
            jnp.where(ge_t, mid_t, hi_t),
        )

    init = (jnp.int32(-1), jnp.int32(0x7FFFFFFF), jnp.int32(-1), jnp.int32(0x7FFFFFFF))
    _, hi_b, _, hi_t = jax.lax.fori_loop(0, 31, body, init)
    return hi_b, hi_t


def _apply_pert(w_ref, hi_b, hi_t, out_dtype):
    b = jax.lax.bitcast_convert_type(w_ref[...], jnp.int32) & _ABS_MASK
    bot = (b <= hi_b).astype(jnp.float32)
    top = (b >= hi_t).astype(jnp.float32)
    return (w_ref[...] * (1.0 + _SCALE * (bot - top))).astype(out_dtype)


def _clean_body(
    x_ref, g_ref, bt_ref, wr_ref, br_ref,
    w1_ref, b1_ref, w2_ref, b2_ref,
    oa_ref, p8_ref,
):
    xb = x_ref[...]
    m = jnp.mean(xb, axis=-1, keepdims=True)
    xc = xb - m
    v = jnp.mean(xc * xc, axis=-1, keepdims=True)
    h = xc * jax.lax.rsqrt(v + 1e-5) * g_ref[...] + bt_ref[...]
    logits = (
        jnp.dot(h.astype(jnp.bfloat16), wr_ref[...].astype(jnp.bfloat16),
                preferred_element_type=jnp.float32)
        + br_ref[...]
    )
    mx = jnp.max(logits, axis=-1, keepdims=True)
    e = jnp.exp(logits - mx)
    p0 = e[:, :1] / jnp.sum(e, axis=-1, keepdims=True)
    hc = jax.nn.gelu(
        jnp.dot(xb.astype(jnp.bfloat16), w1_ref[...],
                preferred_element_type=jnp.float32) + b1_ref[...]
    )
    oa_ref[...] = (
        jnp.dot((p0 * hc).astype(jnp.bfloat16), w2_ref[...],
                preferred_element_type=jnp.float32)
        + p0 * b2_ref[...]
    )
    p8_ref[...] = jnp.broadcast_to(p0, p0.shape[:1] + (_NE,))


def _pert_mlp_body(
    x_ref, p8_ref, oa_ref, w1_ref, b1_ref, w2_ref, b2_ref, thr_ref, o_ref,
    pw1_s, pb1_s, pw2_s, pb2_s,
):
    @pl.when(pl.program_id(0) == 0)
    def _():
        for b_ref, p_ref, n in ((b1_ref, pb1_s, _DF), (b2_ref, pb2_s, _DM)):
            k = max(1, int(n * _SPARSITY / 2))
            hi_b, hi_t = _select2(b_ref, jnp.int32(k), jnp.int32(n - k + 1))
            p_ref[...] = _apply_pert(b_ref, hi_b, hi_t, jnp.float32)
        pw1_s[...] = _apply_pert(w1_ref, thr_ref[0, 0], thr_ref[0, 1], jnp.bfloat16)
        pw2_s[...] = _apply_pert(w2_ref, thr_ref[1, 0], thr_ref[1, 1], jnp.bfloat16)

    pr = 1.0 - p8_ref[:, :1]
    hp = jax.nn.gelu(
        jnp.dot(x_ref[...].astype(jnp.bfloat16), pw1_s[...],
                preferred_element_type=jnp.float32) + pb1_s[...]
    )
    o_ref[...] = (
        oa_ref[...]
        + jnp.dot((pr * hp).astype(jnp.bfloat16), pw2_s[...],
                  preferred_element_type=jnp.float32)
        + pr * pb2_s[...]
    )


def kernel(x, ln_g, ln_b, Wr, br, W1, b1, W2, b2):
    thr = _sc_select(W1.reshape(-1), W2.reshape(-1))

    b1r = b1.reshape(1, _DF)
    b2r = b2.reshape(1, _DM)
    full = lambda shape: pl.BlockSpec(shape, lambda i: (0, 0))
    tok = lambda cols: pl.BlockSpec((_TB, cols), lambda i: (i, 0))

    out_a, p8 = pl.pallas_call(
        _clean_body,
        grid=(_T // _TB,),
        in_specs=[
            tok(_DM),
            full((1, _DM)),
            full((1, _DM)),
            full((_DM, _NE)),
            full((1, _NE)),
            full((_DM, _DF)),
            full((1, _DF)),
            full((_DF, _DM)),
            full((1, _DM)),
        ],
        out_specs=[tok(_DM), tok(_NE)],
        out_shape=[
            jax.ShapeDtypeStruct((_T, _DM), jnp.float32),
            jax.ShapeDtypeStruct((_T, _NE), jnp.float32),
        ],
    )(
        x, ln_g.reshape(1, _DM), ln_b.reshape(1, _DM), Wr, br.reshape(1, _NE),
        W1.astype(jnp.bfloat16), b1r, W2.astype(jnp.bfloat16), b2r,
    )

    out = pl.pallas_call(
        _pert_mlp_body,
        grid=(_T // _TB,),
        in_specs=[
            tok(_DM),
            tok(_NE),
            tok(_DM),
            full((_DM, _DF)),
            full((1, _DF)),
            full((_DF, _DM)),
            full((1, _DM)),
            full((2, 16)),
        ],
        out_specs=tok(_DM),
        out_shape=jax.ShapeDtypeStruct((_T, _DM), jnp.float32),
        scratch_shapes=[
            pltpu.VMEM((_DM, _DF), jnp.bfloat16),
            pltpu.VMEM((1, _DF), jnp.float32),
            pltpu.VMEM((_DF, _DM), jnp.bfloat16),
            pltpu.VMEM((1, _DM), jnp.float32),
        ],
    )(x, p8, out_a, W1, b1r, W2, b2r, thr)
    return out
